# Initial kernel scaffold; baseline (speedup 1.0000x reference)
import jax
import jax.numpy as jnp
from jax.experimental import pallas as pl


def kernel(h, edge_index, edge_attr, ln1_w, ln1_b, Wq, bq, Wk, bk, Wv, bv, We, Wskip, bskip, ga_W1, ga_b1, ga_W2, ga_b2, ga_W3, ga_b3, ln2_w, ln2_b, ff_W1, ff_b1, ff_W2, ff_b2, gf_W1, gf_b1, gf_W2, gf_b2, gf_W3, gf_b3):
    raise NotImplementedError("write your pallas kernel here")


# speedup vs baseline: 2.8111x; 2.8111x over previous
import functools

import jax
import jax.numpy as jnp
import numpy as np
from jax import lax
from jax.experimental import pallas as pl
from jax.experimental.pallas import tpu as pltpu
from jax.experimental.pallas import tpu_sc as plsc

N_ = 10000
E_ = 160000
D_ = 128
H_ = 8
DE_ = 16
NPAD = 10240
EPAD = 163840
NC = 2
NS = 16
NW = NC * NS
EPW = EPAD // NW
ACCW = 144
RSQD = float(1.0 / np.sqrt(D_))
NEG = -1e9

_MESH = functools.partial(
    plsc.VectorSubcoreMesh, core_axis_name="c", subcore_axis_name="s")


def _wt_body(wq, wk, we, bq, bk,
             acat, pcat, wecat, avec, bvec, gvec, cconst):
    dn = (((1,), (1,)), ((), ()))
    for h in range(H_):
        wq_h = wq[:, h * D_:(h + 1) * D_]
        wk_h = wk[:, h * D_:(h + 1) * D_]
        we_h = we[:, h * D_:(h + 1) * D_]
        bq_h = bq[:, h * D_:(h + 1) * D_]
        bk_h = bk[:, h * D_:(h + 1) * D_]
        acat[:, h * D_:(h + 1) * D_] = lax.dot_general(
            wq_h, wk_h, dn, preferred_element_type=jnp.float32)
        pcat[:, h * DE_:(h + 1) * DE_] = lax.dot_general(
            wq_h, we_h, dn, preferred_element_type=jnp.float32)
        wecat[h * DE_:(h + 1) * DE_, :] = we_h
        avec[:, h:h + 1] = lax.dot_general(
            wq_h, bk_h, dn, preferred_element_type=jnp.float32)
        bvec[:, h:h + 1] = lax.dot_general(
            wk_h, bq_h, dn, preferred_element_type=jnp.float32)
        gvec[:, h:h + 1] = lax.dot_general(
            we_h, bq_h, dn, preferred_element_type=jnp.float32)
        cconst[:, h:h + 1] = jnp.sum(bq_h * bk_h, axis=1, keepdims=True)


def _wt_call(Wq, Wk, We, bq2, bk2):
    full = lambda shape: pl.BlockSpec(shape, lambda: (0, 0))
    return pl.pallas_call(
        _wt_body,
        grid=(),
        in_specs=[full((D_, H_ * D_)), full((D_, H_ * D_)), full((DE_, H_ * D_)),
                  full((1, H_ * D_)), full((1, H_ * D_))],
        out_specs=[full((D_, H_ * D_)), full((D_, H_ * DE_)), full((H_ * DE_, D_)),
                   full((D_, H_)), full((D_, H_)), full((DE_, H_)), full((1, H_))],
        out_shape=[jax.ShapeDtypeStruct((D_, H_ * D_), jnp.float32),
                   jax.ShapeDtypeStruct((D_, H_ * DE_), jnp.float32),
                   jax.ShapeDtypeStruct((H_ * DE_, D_), jnp.float32),
                   jax.ShapeDtypeStruct((D_, H_), jnp.float32),
                   jax.ShapeDtypeStruct((D_, H_), jnp.float32),
                   jax.ShapeDtypeStruct((DE_, H_), jnp.float32),
                   jax.ShapeDtypeStruct((1, H_), jnp.float32)],
    )(Wq, Wk, We, bq2, bk2)


def _hnv_body(h_ref, lnw, lnb, wv, bv, hn_ref, v_ref):
    x = h_ref[...]
    mu = jnp.mean(x, axis=1, keepdims=True)
    var = jnp.mean((x - mu) ** 2, axis=1, keepdims=True)
    hn = (x - mu) / jnp.sqrt(var + 1e-5) * lnw[...] + lnb[...]
    hn_ref[...] = hn
    v_ref[...] = jnp.dot(hn, wv[...], preferred_element_type=jnp.float32) + bv[...]


def _hnv_call(h, lnw2, lnb2, Wv, bv2):
    BN = 400
    grid = (N_ // BN,)
    row = lambda shape: pl.BlockSpec(shape, lambda i: (i, 0))
    full = lambda shape: pl.BlockSpec(shape, lambda i: (0, 0))
    return pl.pallas_call(
        _hnv_body,
        grid=grid,
        in_specs=[row((BN, D_)), full((1, D_)), full((1, D_)),
                  full((D_, H_ * D_)), full((1, H_ * D_))],
        out_specs=[row((BN, D_)), row((BN, H_ * D_))],
        out_shape=[jax.ShapeDtypeStruct((N_, D_), jnp.float32),
                   jax.ShapeDtypeStruct((N_, H_ * D_), jnp.float32)],
    )(h, lnw2, lnb2, Wv, bv2)


_BE = 512


def _alpha_body(hnd_ref, hns_ref, ea_ref, acat, pcat, avec, bvec, gvec, cconst,
                out_ref):
    hnd = hnd_ref[...]
    hns = hns_ref[...]
    ea = ea_ref[...]
    cols = []
    for h in range(H_):
        t1 = jnp.dot(hnd, acat[:, h * D_:(h + 1) * D_],
                     preferred_element_type=jnp.float32)
        a = jnp.sum(t1 * hns, axis=1, keepdims=True)
        t2 = jnp.dot(hnd, pcat[:, h * DE_:(h + 1) * DE_],
                     preferred_element_type=jnp.float32)
        a = a + jnp.sum(t2 * ea, axis=1, keepdims=True)
        cols.append(a)
    al = jnp.concatenate(cols, axis=1)
    al = (al
          + jnp.dot(hnd, avec[...], preferred_element_type=jnp.float32)
          + jnp.dot(hns, bvec[...], preferred_element_type=jnp.float32)
          + jnp.dot(ea, gvec[...], preferred_element_type=jnp.float32)
          + cconst[...])
    al = al * RSQD
    al = jnp.concatenate([al, jnp.full((_BE, H_), NEG, jnp.float32)], axis=1)
    i = pl.program_id(0)
    rowid = i * _BE + lax.broadcasted_iota(jnp.int32, (_BE, 1), 0)
    out_ref[...] = jnp.where(rowid < E_, al, NEG)


def _alpha_call(hnd, hns, eap, acat, pcat, avec, bvec, gvec, cconst):
    grid = (EPAD // _BE,)
    row = lambda shape: pl.BlockSpec(shape, lambda i: (i, 0))
    full = lambda shape: pl.BlockSpec(shape, lambda i: (0, 0))
    return pl.pallas_call(
        _alpha_body,
        grid=grid,
        in_specs=[row((_BE, D_)), row((_BE, D_)), row((_BE, DE_)),
                  full((D_, H_ * D_)), full((D_, H_ * DE_)),
                  full((D_, H_)), full((D_, H_)), full((DE_, H_)), full((1, H_))],
        out_specs=row((_BE, 2 * H_)),
        out_shape=jax.ShapeDtypeStruct((EPAD, 2 * H_), jnp.float32),
    )(hnd, hns, eap, acat, pcat, avec, bvec, gvec, cconst)


def _norm_body(acc0, acc1, wecat, invd_ref, ec_ref):
    den = acc0[:, D_:D_ + H_] + acc1[:, D_:D_ + H_]
    inv = 1.0 / (den + 1e-16)
    t = acc0[:, 0:D_] + acc1[:, 0:D_]
    parts = [t[:, h * DE_:(h + 1) * DE_] * inv[:, h:h + 1] for h in range(H_)]
    ts = jnp.concatenate(parts, axis=1)
    ec_ref[...] = jnp.dot(ts, wecat[...], preferred_element_type=jnp.float32)
    invd_ref[...] = jnp.concatenate([inv, jnp.zeros_like(inv)], axis=1)


def _norm_call(acc0, acc1, wecat):
    BN = 512
    grid = (NPAD // BN,)
    row = lambda shape: pl.BlockSpec(shape, lambda i: (i, 0))
    full = lambda shape: pl.BlockSpec(shape, lambda i: (0, 0))
    return pl.pallas_call(
        _norm_body,
        grid=grid,
        in_specs=[row((BN, ACCW)), row((BN, ACCW)), full((H_ * DE_, D_))],
        out_specs=[row((BN, 2 * H_)), row((BN, D_))],
        out_shape=[jax.ShapeDtypeStruct((NPAD, 2 * H_), jnp.float32),
                   jax.ShapeDtypeStruct((NPAD, D_), jnp.float32)],
    )(acc0, acc1, wecat)


def _final_body(hn_ref, o0_ref, o1_ref, ec_ref, wskip, bskip,
                gaW1, gab1, gaW2, gab2, gaW3r, gab3,
                ln2w, ln2b, ffW1, ffb1, ffW2, ffb2,
                gfW1, gfb1, gfW2, gfb2, gfW3r, gfb3, out_ref):
    hn = hn_ref[...]
    out = ((o0_ref[...] + o1_ref[...] + ec_ref[...]) * (1.0 / H_)
           + jnp.dot(hn, wskip[...], preferred_element_type=jnp.float32)
           + bskip[...])

    def gate(u, v, W1, b1, W2, b2, W3r, b3):
        z = jnp.concatenate([u, v, u - v], axis=1)
        a = jnp.dot(z, W1[...], preferred_element_type=jnp.float32) + b1[...]
        a = a * jax.nn.sigmoid(a)
        a = jnp.dot(a, W2[...], preferred_element_type=jnp.float32) + b2[...]
        a = a * jax.nn.sigmoid(a)
        g = jnp.sum(a * W3r[...], axis=1, keepdims=True) + b3[...]
        g = jax.nn.sigmoid(g)
        return g * u + (1 - g) * v

    h1 = gate(hn, out, gaW1, gab1, gaW2, gab2, gaW3r, gab3)
    mu = jnp.mean(h1, axis=1, keepdims=True)
    var = jnp.mean((h1 - mu) ** 2, axis=1, keepdims=True)
    h2 = (h1 - mu) / jnp.sqrt(var + 1e-5) * ln2w[...] + ln2b[...]
    ff = jnp.dot(h2, ffW1[...], preferred_element_type=jnp.float32) + ffb1[...]
    ff = ff * jax.nn.sigmoid(ff)
    ff = jnp.dot(ff, ffW2[...], preferred_element_type=jnp.float32) + ffb2[...]
    out_ref[...] = gate(h2, ff, gfW1, gfb1, gfW2, gfb2, gfW3r, gfb3)


def _final_call(hn, o0, o1, ec, Wskip, bskip2, ga, ln2w2, ln2b2, ff, gf):
    BN = 400
    grid = (N_ // BN,)
    row = lambda shape: pl.BlockSpec(shape, lambda i: (i, 0))
    full = lambda shape: pl.BlockSpec(shape, lambda i: (0, 0))
    D3, D32, D34 = 3 * D_, 3 * D_ // 2, 3 * D_ // 4
    in_specs = [row((BN, D_)), row((BN, D_)), row((BN, D_)), row((BN, D_)),
                full((D_, D_)), full((1, D_)),
                full((D3, D32)), full((1, D32)), full((D32, D34)), full((1, D34)),
                full((1, D34)), full((1, 1)),
                full((1, D_)), full((1, D_)),
                full((D_, D_)), full((1, D_)), full((D_, D_)), full((1, D_)),
                full((D3, D32)), full((1, D32)), full((D32, D34)), full((1, D34)),
                full((1, D34)), full((1, 1))]
    return pl.pallas_call(
        _final_body,
        grid=grid,
        in_specs=in_specs,
        out_specs=row((BN, D_)),
        out_shape=jax.ShapeDtypeStruct((N_, D_), jnp.float32),
    )(hn, o0, o1, ec, Wskip, bskip2, *ga, ln2w2, ln2b2, *ff, *gf)


def _sc_gather(hn, srcp, dstp):
    CB = 128
    nch = EPW // CB

    @functools.partial(
        pl.kernel,
        out_type=(jax.ShapeDtypeStruct((EPAD, D_), jnp.float32),
                  jax.ShapeDtypeStruct((EPAD, D_), jnp.float32)),
        mesh=_MESH(),
        compiler_params=pltpu.CompilerParams(use_tc_tiling_on_sc=False),
        scratch_types=[pltpu.VMEM((CB,), jnp.int32), pltpu.VMEM((CB,), jnp.int32),
                       pltpu.VMEM((CB, D_), jnp.float32),
                       pltpu.VMEM((CB, D_), jnp.float32),
                       pltpu.SemaphoreType.DMA, pltpu.SemaphoreType.DMA],
    )
    def k(hn_hbm, src_hbm, dst_hbm, hns_hbm, hnd_hbm,
          sidx, didx, srows, drows, sem1, sem2):
        wid = lax.axis_index("s") * NC + lax.axis_index("c")

        def body(j, carry):
            base = wid * EPW + j * CB
            pltpu.sync_copy(src_hbm.at[pl.ds(base, CB)], sidx)
            pltpu.sync_copy(dst_hbm.at[pl.ds(base, CB)], didx)
            c1 = pltpu.async_copy(hn_hbm.at[sidx], srows, sem1)
            c2 = pltpu.async_copy(hn_hbm.at[didx], drows, sem2)
            c1.wait()
            c2.wait()
            pltpu.sync_copy(srows, hns_hbm.at[pl.ds(base, CB)])
            pltpu.sync_copy(drows, hnd_hbm.at[pl.ds(base, CB)])
            return carry

        lax.fori_loop(0, nch, body, 0)

    return k(hn, srcp, dstp)


def _sc_p1(alpha, eap, dstp):
    CB = 128
    nch = EPW // CB
    rpt = NPAD // NS

    @functools.partial(
        pl.kernel,
        out_type=jax.ShapeDtypeStruct((NC * NPAD, ACCW), jnp.float32),
        mesh=_MESH(),
        compiler_params=pltpu.CompilerParams(use_tc_tiling_on_sc=False),
        scratch_types=[pltpu.VMEM((CB, 2 * H_), jnp.float32),
                       pltpu.VMEM((CB, DE_), jnp.float32),
                       pltpu.VMEM((CB,), jnp.int32),
                       pltpu.VMEM((CB, ACCW), jnp.float32),
                       pltpu.VMEM_SHARED((NPAD, ACCW), jnp.float32)],
    )
    def k(alpha_hbm, ea_hbm, dst_hbm, out_hbm, abuf, eabuf, didx, payload, acc):
        cid = lax.axis_index("c")
        sid = lax.axis_index("s")
        wid = sid * NC + cid

        def zrow(i, c):
            for j in range(ACCW // 16):
                payload[i, pl.ds(j * 16, 16)] = jnp.zeros((16,), jnp.float32)
            return c
        lax.fori_loop(0, CB, zrow, 0)

        def zacc(i, c):
            pltpu.sync_copy(payload, acc.at[pl.ds(sid * rpt + i * CB, CB)])
            return c
        lax.fori_loop(0, rpt // CB, zacc, 0)
        plsc.subcore_barrier()

        def chunk(j, carry):
            base = wid * EPW + j * CB
            pltpu.sync_copy(alpha_hbm.at[pl.ds(base, CB)], abuf)
            pltpu.sync_copy(ea_hbm.at[pl.ds(base, CB)], eabuf)
            pltpu.sync_copy(dst_hbm.at[pl.ds(base, CB)], didx)

            def edge(e, c2):
                ex16 = jnp.exp(abuf[e, pl.ds(0, 16)])
                payload[e, pl.ds(D_, 16)] = ex16
                earow = eabuf[e, pl.ds(0, DE_)]
                for h in range(H_):
                    payload[e, pl.ds(h * DE_, DE_)] = (
                        jnp.full((16,), ex16[h]) * earow)
                return c2

            lax.fori_loop(0, CB, edge, 0)
            pltpu.sync_copy(payload, acc.at[didx], add=True)
            return carry

        lax.fori_loop(0, nch, chunk, 0)
        plsc.subcore_barrier()

        def wout(i, c):
            r0 = sid * rpt + i * CB
            pltpu.sync_copy(acc.at[pl.ds(r0, CB)],
                            out_hbm.at[pl.ds(cid * NPAD + r0, CB)])
            return c
        lax.fori_loop(0, rpt // CB, wout, 0)

    return k(alpha, eap, dstp)


def _sc_p2(alpha, srcp, dstp, V, invd):
    CB = 32
    nch = EPW // CB
    rpt = NPAD // NS

    @functools.partial(
        pl.kernel,
        out_type=jax.ShapeDtypeStruct((NC * NPAD, D_), jnp.float32),
        mesh=_MESH(),
        compiler_params=pltpu.CompilerParams(use_tc_tiling_on_sc=False),
        scratch_types=[pltpu.VMEM((CB, 2 * H_), jnp.float32),
                       pltpu.VMEM((CB, 2 * H_), jnp.float32),
                       pltpu.VMEM((CB,), jnp.int32),
                       pltpu.VMEM((CB,), jnp.int32),
                       pltpu.VMEM((CB, H_ * D_), jnp.float32),
                       pltpu.VMEM((CB, D_), jnp.float32),
                       pltpu.VMEM_SHARED((NPAD, D_), jnp.float32),
                       pltpu.SemaphoreType.DMA, pltpu.SemaphoreType.DMA],
    )
    def k(alpha_hbm, src_hbm, dst_hbm, v_hbm, invd_hbm, out_hbm,
          abuf, ivbuf, sidx, didx, vrows, wpay, acc, sem1, sem2):
        cid = lax.axis_index("c")
        sid = lax.axis_index("s")
        wid = sid * NC + cid

        def zrow(i, c):
            for j in range(D_ // 16):
                wpay[i, pl.ds(j * 16, 16)] = jnp.zeros((16,), jnp.float32)
            return c
        lax.fori_loop(0, CB, zrow, 0)

        def zacc(i, c):
            pltpu.sync_copy(wpay, acc.at[pl.ds(sid * rpt + i * CB, CB)])
            return c
        lax.fori_loop(0, rpt // CB, zacc, 0)
        plsc.subcore_barrier()

        def chunk(j, carry):
            base = wid * EPW + j * CB
            pltpu.sync_copy(src_hbm.at[pl.ds(base, CB)], sidx)
            pltpu.sync_copy(dst_hbm.at[pl.ds(base, CB)], didx)
            pltpu.sync_copy(alpha_hbm.at[pl.ds(base, CB)], abuf)
            gv = pltpu.async_copy(v_hbm.at[sidx], vrows, sem1)
            gi = pltpu.async_copy(invd_hbm.at[didx], ivbuf, sem2)
            gi.wait()
            gv.wait()

            def edge(e, c2):
                attn16 = (jnp.exp(abuf[e, pl.ds(0, 16)])
                          * ivbuf[e, pl.ds(0, 16)])
                accs = [jnp.zeros((16,), jnp.float32)
                        for _ in range(D_ // 16)]
                for h in range(H_):
                    avv = jnp.full((16,), attn16[h])
                    for dv in range(D_ // 16):
                        seg = vrows[e, pl.ds(h * D_ + dv * 16, 16)]
                        accs[dv] = accs[dv] + avv * seg
                for dv in range(D_ // 16):
                    wpay[e, pl.ds(dv * 16, 16)] = accs[dv]
                return c2

            lax.fori_loop(0, CB, edge, 0)
            pltpu.sync_copy(wpay, acc.at[didx], add=True)
            return carry

        lax.fori_loop(0, nch, chunk, 0)
        plsc.subcore_barrier()

        def wout(i, c):
            r0 = sid * rpt + i * CB
            pltpu.sync_copy(acc.at[pl.ds(r0, CB)],
                            out_hbm.at[pl.ds(cid * NPAD + r0, CB)])
            return c
        lax.fori_loop(0, rpt // CB, wout, 0)

    return k(alpha, srcp, dstp, V, invd)


def kernel(h, edge_index, edge_attr, ln1_w, ln1_b, Wq, bq, Wk, bk, Wv, bv, We,
           Wskip, bskip, ga_W1, ga_b1, ga_W2, ga_b2, ga_W3, ga_b3, ln2_w, ln2_b,
           ff_W1, ff_b1, ff_W2, ff_b2, gf_W1, gf_b1, gf_W2, gf_b2, gf_W3, gf_b3):
    pad_e = EPAD - E_
    srcp = jnp.concatenate([edge_index[0], jnp.zeros((pad_e,), jnp.int32)])
    dstp = jnp.concatenate([edge_index[1], jnp.zeros((pad_e,), jnp.int32)])
    eap = jnp.concatenate(
        [edge_attr, jnp.zeros((pad_e, DE_), jnp.float32)], axis=0)

    acat, pcat, wecat, avec, bvec, gvec, cconst = _wt_call(
        Wq, Wk, We, bq.reshape(1, -1), bk.reshape(1, -1))
    hn, V = _hnv_call(h, ln1_w.reshape(1, -1), ln1_b.reshape(1, -1),
                      Wv, bv.reshape(1, -1))
    hns, hnd = _sc_gather(hn, srcp, dstp)
    alpha = _alpha_call(hnd, hns, eap, acat, pcat, avec, bvec, gvec, cconst)
    accf = _sc_p1(alpha, eap, dstp)
    invd, ec = _norm_call(accf[:NPAD], accf[NPAD:], wecat)
    outf = _sc_p2(alpha, srcp, dstp, V, invd)
    ga = (ga_W1, ga_b1.reshape(1, -1), ga_W2, ga_b2.reshape(1, -1),
          ga_W3.reshape(1, -1), ga_b3.reshape(1, -1))
    ff = (ff_W1, ff_b1.reshape(1, -1), ff_W2, ff_b2.reshape(1, -1))
    gf = (gf_W1, gf_b1.reshape(1, -1), gf_W2, gf_b2.reshape(1, -1),
          gf_W3.reshape(1, -1), gf_b3.reshape(1, -1))
    return _final_call(hn, outf[:N_], outf[NPAD:NPAD + N_], ec[:N_],
                       Wskip, bskip.reshape(1, -1), ga,
                       ln2_w.reshape(1, -1), ln2_b.reshape(1, -1), ff, gf)


# speedup vs baseline: 3.1002x; 1.1028x over previous
import functools

import jax
import jax.numpy as jnp
import numpy as np
from jax import lax
from jax.experimental import pallas as pl
from jax.experimental.pallas import tpu as pltpu
from jax.experimental.pallas import tpu_sc as plsc

N_ = 10000
E_ = 160000
D_ = 128
H_ = 8
DE_ = 16
NPAD = 10240
EPAD = 163840
NC = 2
NS = 16
NW = NC * NS
EPW = EPAD // NW
ACCW = 144
RSQD = float(1.0 / np.sqrt(D_))
NEG = -1e9

_MESH = functools.partial(
    plsc.VectorSubcoreMesh, core_axis_name="c", subcore_axis_name="s")


def _wt_body(wq, wk, we, bq, bk,
             acat, pcat, wecat, avec, bvec, gvec, cconst):
    dn = (((1,), (1,)), ((), ()))
    for h in range(H_):
        wq_h = wq[:, h * D_:(h + 1) * D_]
        wk_h = wk[:, h * D_:(h + 1) * D_]
        we_h = we[:, h * D_:(h + 1) * D_]
        bq_h = bq[:, h * D_:(h + 1) * D_]
        bk_h = bk[:, h * D_:(h + 1) * D_]
        acat[:, h * D_:(h + 1) * D_] = lax.dot_general(
            wq_h, wk_h, dn, preferred_element_type=jnp.float32).astype(
                jnp.bfloat16)
        pcat[:, h * DE_:(h + 1) * DE_] = lax.dot_general(
            wq_h, we_h, dn, preferred_element_type=jnp.float32).astype(
                jnp.bfloat16)
        wecat[h * DE_:(h + 1) * DE_, :] = we_h
        avec[:, h:h + 1] = lax.dot_general(
            wq_h, bk_h, dn, preferred_element_type=jnp.float32)
        bvec[:, h:h + 1] = lax.dot_general(
            wk_h, bq_h, dn, preferred_element_type=jnp.float32)
        gvec[:, h:h + 1] = lax.dot_general(
            we_h, bq_h, dn, preferred_element_type=jnp.float32)
        cconst[:, h:h + 1] = jnp.sum(bq_h * bk_h, axis=1, keepdims=True)


def _wt_call(Wq, Wk, We, bq2, bk2):
    full = lambda shape: pl.BlockSpec(shape, lambda: (0, 0))
    return pl.pallas_call(
        _wt_body,
        grid=(),
        in_specs=[full((D_, H_ * D_)), full((D_, H_ * D_)), full((DE_, H_ * D_)),
                  full((1, H_ * D_)), full((1, H_ * D_))],
        out_specs=[full((D_, H_ * D_)), full((D_, H_ * DE_)), full((H_ * DE_, D_)),
                   full((D_, H_)), full((D_, H_)), full((DE_, H_)), full((1, H_))],
        out_shape=[jax.ShapeDtypeStruct((D_, H_ * D_), jnp.bfloat16),
                   jax.ShapeDtypeStruct((D_, H_ * DE_), jnp.bfloat16),
                   jax.ShapeDtypeStruct((H_ * DE_, D_), jnp.float32),
                   jax.ShapeDtypeStruct((D_, H_), jnp.float32),
                   jax.ShapeDtypeStruct((D_, H_), jnp.float32),
                   jax.ShapeDtypeStruct((DE_, H_), jnp.float32),
                   jax.ShapeDtypeStruct((1, H_), jnp.float32)],
    )(Wq, Wk, We, bq2, bk2)


def _hnv_body(h_ref, lnw, lnb, wv, bv, hn_ref, hnb_ref, v_ref):
    x = h_ref[...]
    mu = jnp.mean(x, axis=1, keepdims=True)
    var = jnp.mean((x - mu) ** 2, axis=1, keepdims=True)
    hn = (x - mu) / jnp.sqrt(var + 1e-5) * lnw[...] + lnb[...]
    hn_ref[...] = hn
    hnb_ref[...] = hn.astype(jnp.bfloat16)
    v = jnp.dot(hn, wv[...], preferred_element_type=jnp.float32) + bv[...]
    v_ref[...] = v.astype(jnp.bfloat16)


def _hnv_call(h, lnw2, lnb2, Wv, bv2):
    BN = 400
    grid = (N_ // BN,)
    row = lambda shape: pl.BlockSpec(shape, lambda i: (i, 0))
    full = lambda shape: pl.BlockSpec(shape, lambda i: (0, 0))
    return pl.pallas_call(
        _hnv_body,
        grid=grid,
        in_specs=[row((BN, D_)), full((1, D_)), full((1, D_)),
                  full((D_, H_ * D_)), full((1, H_ * D_))],
        out_specs=[row((BN, D_)), row((BN, D_)), row((BN, H_ * D_))],
        out_shape=[jax.ShapeDtypeStruct((N_, D_), jnp.float32),
                   jax.ShapeDtypeStruct((N_, D_), jnp.bfloat16),
                   jax.ShapeDtypeStruct((N_, H_ * D_), jnp.bfloat16)],
    )(h, lnw2, lnb2, Wv, bv2)


_BE = 512


def _alpha_body(hnd_ref, hns_ref, ea_ref, acat, pcat, avec, bvec, gvec, cconst,
                out_ref):
    hnd = hnd_ref[...]
    hns = hns_ref[...]
    hndf = hnd.astype(jnp.float32)
    hnsf = hns.astype(jnp.float32)
    ea = ea_ref[...]
    cols = []
    for h in range(H_):
        t1 = jnp.dot(hnd, acat[:, h * D_:(h + 1) * D_],
                     preferred_element_type=jnp.float32)
        a = jnp.sum(t1 * hnsf, axis=1, keepdims=True)
        t2 = jnp.dot(hnd, pcat[:, h * DE_:(h + 1) * DE_],
                     preferred_element_type=jnp.float32)
        a = a + jnp.sum(t2 * ea, axis=1, keepdims=True)
        cols.append(a)
    al = jnp.concatenate(cols, axis=1)
    al = (al
          + jnp.dot(hndf, avec[...], preferred_element_type=jnp.float32)
          + jnp.dot(hnsf, bvec[...], preferred_element_type=jnp.float32)
          + jnp.dot(ea, gvec[...], preferred_element_type=jnp.float32)
          + cconst[...])
    al = al * RSQD
    al = jnp.concatenate([al, jnp.full((_BE, H_), NEG, jnp.float32)], axis=1)
    i = pl.program_id(0)
    rowid = i * _BE + lax.broadcasted_iota(jnp.int32, (_BE, 1), 0)
    out_ref[...] = jnp.where(rowid < E_, al, NEG)


def _alpha_call(hnd, hns, eap, acat, pcat, avec, bvec, gvec, cconst):
    grid = (EPAD // _BE,)
    row = lambda shape: pl.BlockSpec(shape, lambda i: (i, 0))
    full = lambda shape: pl.BlockSpec(shape, lambda i: (0, 0))
    return pl.pallas_call(
        _alpha_body,
        grid=grid,
        in_specs=[row((_BE, D_)), row((_BE, D_)), row((_BE, DE_)),
                  full((D_, H_ * D_)), full((D_, H_ * DE_)),
                  full((D_, H_)), full((D_, H_)), full((DE_, H_)),
                  full((1, H_))],
        out_specs=row((_BE, 2 * H_)),
        out_shape=jax.ShapeDtypeStruct((EPAD, 2 * H_), jnp.float32),
    )(hnd, hns, eap, acat, pcat, avec, bvec, gvec, cconst)


def _norm_body(acc0, acc1, wecat, invd_ref, ec_ref):
    den = acc0[:, D_:D_ + H_] + acc1[:, D_:D_ + H_]
    inv = 1.0 / (den + 1e-16)
    t = acc0[:, 0:D_] + acc1[:, 0:D_]
    parts = [t[:, h * DE_:(h + 1) * DE_] * inv[:, h:h + 1] for h in range(H_)]
    ts = jnp.concatenate(parts, axis=1)
    ec_ref[...] = jnp.dot(ts, wecat[...], preferred_element_type=jnp.float32)
    invd_ref[...] = jnp.concatenate([inv, jnp.zeros_like(inv)], axis=1)


def _norm_call(acc0, acc1, wecat):
    BN = 512
    grid = (NPAD // BN,)
    row = lambda shape: pl.BlockSpec(shape, lambda i: (i, 0))
    full = lambda shape: pl.BlockSpec(shape, lambda i: (0, 0))
    return pl.pallas_call(
        _norm_body,
        grid=grid,
        in_specs=[row((BN, ACCW)), row((BN, ACCW)), full((H_ * DE_, D_))],
        out_specs=[row((BN, 2 * H_)), row((BN, D_))],
        out_shape=[jax.ShapeDtypeStruct((NPAD, 2 * H_), jnp.float32),
                   jax.ShapeDtypeStruct((NPAD, D_), jnp.float32)],
    )(acc0, acc1, wecat)


def _final_body(hn_ref, o0_ref, o1_ref, ec_ref, wskip, bskip,
                gaW1, gab1, gaW2, gab2, gaW3r, gab3,
                ln2w, ln2b, ffW1, ffb1, ffW2, ffb2,
                gfW1, gfb1, gfW2, gfb2, gfW3r, gfb3, out_ref):
    hn = hn_ref[...]
    p = lax.broadcasted_iota(jnp.int32, (D_, D_), 0)
    c = lax.broadcasted_iota(jnp.int32, (D_, D_), 1)
    tgt = ((p >> 5) << 5) + 2 * (p & 15) + ((p >> 4) & 1)
    perm = (c == tgt).astype(jnp.float32)
    op = jnp.dot(o0_ref[...] + o1_ref[...], perm,
                 preferred_element_type=jnp.float32)
    out = ((op + ec_ref[...]) * (1.0 / H_)
           + jnp.dot(hn, wskip[...], preferred_element_type=jnp.float32)
           + bskip[...])

    def gate(u, v, W1, b1, W2, b2, W3r, b3):
        z = jnp.concatenate([u, v, u - v], axis=1)
        a = jnp.dot(z, W1[...], preferred_element_type=jnp.float32) + b1[...]
        a = a * jax.nn.sigmoid(a)
        a = jnp.dot(a, W2[...], preferred_element_type=jnp.float32) + b2[...]
        a = a * jax.nn.sigmoid(a)
        g = jnp.sum(a * W3r[...], axis=1, keepdims=True) + b3[...]
        g = jax.nn.sigmoid(g)
        return g * u + (1 - g) * v

    h1 = gate(hn, out, gaW1, gab1, gaW2, gab2, gaW3r, gab3)
    mu = jnp.mean(h1, axis=1, keepdims=True)
    var = jnp.mean((h1 - mu) ** 2, axis=1, keepdims=True)
    h2 = (h1 - mu) / jnp.sqrt(var + 1e-5) * ln2w[...] + ln2b[...]
    ff = jnp.dot(h2, ffW1[...], preferred_element_type=jnp.float32) + ffb1[...]
    ff = ff * jax.nn.sigmoid(ff)
    ff = jnp.dot(ff, ffW2[...], preferred_element_type=jnp.float32) + ffb2[...]
    out_ref[...] = gate(h2, ff, gfW1, gfb1, gfW2, gfb2, gfW3r, gfb3)


def _final_call(hn, o0, o1, ec, Wskip, bskip2, ga, ln2w2, ln2b2, ff, gf):
    BN = 400
    grid = (N_ // BN,)
    row = lambda shape: pl.BlockSpec(shape, lambda i: (i, 0))
    full = lambda shape: pl.BlockSpec(shape, lambda i: (0, 0))
    D3, D32, D34 = 3 * D_, 3 * D_ // 2, 3 * D_ // 4
    in_specs = [row((BN, D_)), row((BN, D_)), row((BN, D_)), row((BN, D_)),
                full((D_, D_)), full((1, D_)),
                full((D3, D32)), full((1, D32)), full((D32, D34)), full((1, D34)),
                full((1, D34)), full((1, 1)),
                full((1, D_)), full((1, D_)),
                full((D_, D_)), full((1, D_)), full((D_, D_)), full((1, D_)),
                full((D3, D32)), full((1, D32)), full((D32, D34)), full((1, D34)),
                full((1, D34)), full((1, 1))]
    return pl.pallas_call(
        _final_body,
        grid=grid,
        in_specs=in_specs,
        out_specs=row((BN, D_)),
        out_shape=jax.ShapeDtypeStruct((N_, D_), jnp.float32),
    )(hn, o0, o1, ec, Wskip, bskip2, *ga, ln2w2, ln2b2, *ff, *gf)


def _sc_gather(hnb, srcp, dstp):
    CB = 128
    nch = EPW // CB

    @functools.partial(
        pl.kernel,
        out_type=(jax.ShapeDtypeStruct((EPAD, D_), jnp.bfloat16),
                  jax.ShapeDtypeStruct((EPAD, D_), jnp.bfloat16)),
        mesh=_MESH(),
        compiler_params=pltpu.CompilerParams(use_tc_tiling_on_sc=False, needs_layout_passes=False),
        scratch_types=[pltpu.VMEM((2, CB), jnp.int32),
                       pltpu.VMEM((2, CB), jnp.int32),
                       pltpu.VMEM((2, CB, D_), jnp.bfloat16),
                       pltpu.VMEM((2, CB, D_), jnp.bfloat16),
                       pltpu.SemaphoreType.DMA, pltpu.SemaphoreType.DMA,
                       pltpu.SemaphoreType.DMA, pltpu.SemaphoreType.DMA],
    )
    def k(hn_hbm, src_hbm, dst_hbm, hns_hbm, hnd_hbm,
          sidx, didx, srows, drows, s_s0, s_s1, s_d0, s_d1):
        wid = lax.axis_index("s") * NC + lax.axis_index("c")
        ssems = (s_s0, s_s1)
        dsems = (s_d0, s_d1)

        def start(j, b):
            base = wid * EPW + j * CB
            pltpu.sync_copy(src_hbm.at[pl.ds(base, CB)], sidx.at[b])
            pltpu.sync_copy(dst_hbm.at[pl.ds(base, CB)], didx.at[b])
            pltpu.async_copy(hn_hbm.at[sidx.at[b]], srows.at[b], ssems[b])
            pltpu.async_copy(hn_hbm.at[didx.at[b]], drows.at[b], dsems[b])

        def drain(j, b):
            base = wid * EPW + j * CB
            pltpu.make_async_copy(hn_hbm.at[sidx.at[b]], srows.at[b],
                                  ssems[b]).wait()
            pltpu.make_async_copy(hn_hbm.at[didx.at[b]], drows.at[b],
                                  dsems[b]).wait()
            pltpu.sync_copy(srows.at[b], hns_hbm.at[pl.ds(base, CB)])
            pltpu.sync_copy(drows.at[b], hnd_hbm.at[pl.ds(base, CB)])

        start(0, 0)

        def body(p, carry):
            start(2 * p + 1, 1)
            drain(2 * p, 0)

            @pl.when(p + 1 < nch // 2)
            def _():
                start(2 * p + 2, 0)
            drain(2 * p + 1, 1)
            return carry

        lax.fori_loop(0, nch // 2, body, 0)

    return k(hnb, srcp, dstp)


def _sc_p1(alpha, eap, dstp):
    CB = 128
    nch = EPW // CB
    rpt = NPAD // NS

    @functools.partial(
        pl.kernel,
        out_type=jax.ShapeDtypeStruct((NC * NPAD, ACCW), jnp.float32),
        mesh=_MESH(),
        compiler_params=pltpu.CompilerParams(use_tc_tiling_on_sc=False, needs_layout_passes=False),
        scratch_types=[pltpu.VMEM((CB, 2 * H_), jnp.float32),
                       pltpu.VMEM((CB, DE_), jnp.float32),
                       pltpu.VMEM((CB,), jnp.int32),
                       pltpu.VMEM((CB, ACCW), jnp.float32),
                       pltpu.VMEM_SHARED((NPAD, ACCW), jnp.float32)],
    )
    def k(alpha_hbm, ea_hbm, dst_hbm, out_hbm, abuf, eabuf, didx, payload, acc):
        cid = lax.axis_index("c")
        sid = lax.axis_index("s")
        wid = sid * NC + cid

        def zrow(i, c):
            for j in range(ACCW // 16):
                payload[i, pl.ds(j * 16, 16)] = jnp.zeros((16,), jnp.float32)
            return c
        lax.fori_loop(0, CB, zrow, 0)

        def zacc(i, c):
            pltpu.sync_copy(payload, acc.at[pl.ds(sid * rpt + i * CB, CB)])
            return c
        lax.fori_loop(0, rpt // CB, zacc, 0)
        plsc.subcore_barrier()

        def chunk(j, carry):
            base = wid * EPW + j * CB
            pltpu.sync_copy(alpha_hbm.at[pl.ds(base, CB)], abuf)
            pltpu.sync_copy(ea_hbm.at[pl.ds(base, CB)], eabuf)
            pltpu.sync_copy(dst_hbm.at[pl.ds(base, CB)], didx)

            def edge(e, c2):
                ex16 = jnp.exp(abuf[e, pl.ds(0, 16)])
                payload[e, pl.ds(D_, 16)] = ex16
                earow = eabuf[e, pl.ds(0, DE_)]
                for h in range(H_):
                    payload[e, pl.ds(h * DE_, DE_)] = (
                        jnp.full((16,), ex16[h]) * earow)
                return c2

            lax.fori_loop(0, CB, edge, 0)
            pltpu.sync_copy(payload, acc.at[didx], add=True)
            return carry

        lax.fori_loop(0, nch, chunk, 0)
        plsc.subcore_barrier()

        def wout(i, c):
            r0 = sid * rpt + i * CB
            pltpu.sync_copy(acc.at[pl.ds(r0, CB)],
                            out_hbm.at[pl.ds(cid * NPAD + r0, CB)])
            return c
        lax.fori_loop(0, rpt // CB, wout, 0)

    return k(alpha, eap, dstp)


def _sc_p2(alpha, srcp, dstp, Vb, invd):
    CB = 32
    nch = EPW // CB
    rpt = NPAD // NS

    @functools.partial(
        pl.kernel,
        out_type=jax.ShapeDtypeStruct((NC * NPAD, D_), jnp.float32),
        mesh=_MESH(),
        compiler_params=pltpu.CompilerParams(use_tc_tiling_on_sc=False, needs_layout_passes=False),
        scratch_types=[pltpu.VMEM((2, CB, 2 * H_), jnp.float32),
                       pltpu.VMEM((2, CB, 2 * H_), jnp.float32),
                       pltpu.VMEM((2, CB), jnp.int32),
                       pltpu.VMEM((2, CB), jnp.int32),
                       pltpu.VMEM((2, CB, H_ * D_), jnp.bfloat16),
                       pltpu.VMEM((CB, D_), jnp.float32),
                       pltpu.VMEM_SHARED((NPAD, D_), jnp.float32),
                       pltpu.SemaphoreType.DMA, pltpu.SemaphoreType.DMA,
                       pltpu.SemaphoreType.DMA, pltpu.SemaphoreType.DMA],
    )
    def k(alpha_hbm, src_hbm, dst_hbm, v_hbm, invd_hbm, out_hbm,
          abuf, ivbuf, sidx, didx, vrows, wpay, acc, sv0, sv1, si0, si1):
        cid = lax.axis_index("c")
        sid = lax.axis_index("s")
        wid = sid * NC + cid
        vsems = (sv0, sv1)
        isems = (si0, si1)

        def zrow(i, c):
            for j in range(D_ // 16):
                wpay[i, pl.ds(j * 16, 16)] = jnp.zeros((16,), jnp.float32)
            return c
        lax.fori_loop(0, CB, zrow, 0)

        def zacc(i, c):
            pltpu.sync_copy(wpay, acc.at[pl.ds(sid * rpt + i * CB, CB)])
            return c
        lax.fori_loop(0, rpt // CB, zacc, 0)
        plsc.subcore_barrier()

        def start(j, b):
            base = wid * EPW + j * CB
            pltpu.sync_copy(src_hbm.at[pl.ds(base, CB)], sidx.at[b])
            pltpu.sync_copy(dst_hbm.at[pl.ds(base, CB)], didx.at[b])
            pltpu.sync_copy(alpha_hbm.at[pl.ds(base, CB)], abuf.at[b])
            pltpu.async_copy(v_hbm.at[sidx.at[b]], vrows.at[b], vsems[b])
            pltpu.async_copy(invd_hbm.at[didx.at[b]], ivbuf.at[b], isems[b])

        def process(b):
            pltpu.make_async_copy(
                v_hbm.at[sidx.at[b]], vrows.at[b], vsems[b]).wait()
            pltpu.make_async_copy(
                invd_hbm.at[didx.at[b]], ivbuf.at[b], isems[b]).wait()

            def edge(e, c2):
                attn16 = (jnp.exp(abuf[b, e, pl.ds(0, 16)])
                          * ivbuf[b, e, pl.ds(0, 16)])
                accs = [jnp.zeros((16,), jnp.float32)
                        for _ in range(D_ // 16)]
                for h in range(H_):
                    avv = jnp.full((16,), attn16[h])
                    for g in range(D_ // 32):
                        x32 = vrows[b, e, pl.ds(h * D_ + g * 32, 32)]
                        lo, hi = plsc.unpack(
                            x32, format=plsc.PackFormat.INTERLEAVED)
                        accs[2 * g] = accs[2 * g] + avv * lo
                        accs[2 * g + 1] = accs[2 * g + 1] + avv * hi
                for dv in range(D_ // 16):
                    wpay[e, pl.ds(dv * 16, 16)] = accs[dv]
                return c2

            lax.fori_loop(0, CB, edge, 0)
            pltpu.sync_copy(wpay, acc.at[didx.at[b]], add=True)

        start(0, 0)

        def body(p, carry):
            start(2 * p + 1, 1)
            process(0)

            @pl.when(p + 1 < nch // 2)
            def _():
                start(2 * p + 2, 0)
            process(1)
            return carry

        lax.fori_loop(0, nch // 2, body, 0)
        plsc.subcore_barrier()

        def wout(i, c):
            r0 = sid * rpt + i * CB
            pltpu.sync_copy(acc.at[pl.ds(r0, CB)],
                            out_hbm.at[pl.ds(cid * NPAD + r0, CB)])
            return c
        lax.fori_loop(0, rpt // CB, wout, 0)

    return k(alpha, srcp, dstp, Vb, invd)


def kernel(h, edge_index, edge_attr, ln1_w, ln1_b, Wq, bq, Wk, bk, Wv, bv, We,
           Wskip, bskip, ga_W1, ga_b1, ga_W2, ga_b2, ga_W3, ga_b3, ln2_w, ln2_b,
           ff_W1, ff_b1, ff_W2, ff_b2, gf_W1, gf_b1, gf_W2, gf_b2, gf_W3, gf_b3):
    pad_e = EPAD - E_
    srcp = jnp.concatenate([edge_index[0], jnp.zeros((pad_e,), jnp.int32)])
    dstp = jnp.concatenate([edge_index[1], jnp.zeros((pad_e,), jnp.int32)])
    eap = jnp.concatenate(
        [edge_attr, jnp.zeros((pad_e, DE_), jnp.float32)], axis=0)

    acat, pcat, wecat, avec, bvec, gvec, cconst = _wt_call(
        Wq, Wk, We, bq.reshape(1, -1), bk.reshape(1, -1))
    hn, hnb, Vb = _hnv_call(h, ln1_w.reshape(1, -1), ln1_b.reshape(1, -1),
                            Wv, bv.reshape(1, -1))
    hns, hnd = _sc_gather(hnb, srcp, dstp)
    alpha = _alpha_call(hnd, hns, eap, acat, pcat, avec, bvec, gvec, cconst)
    accf = _sc_p1(alpha, eap, dstp)
    invd, ec = _norm_call(accf[:NPAD], accf[NPAD:], wecat)
    outf = _sc_p2(alpha, srcp, dstp, Vb, invd)
    ga = (ga_W1, ga_b1.reshape(1, -1), ga_W2, ga_b2.reshape(1, -1),
          ga_W3.reshape(1, -1), ga_b3.reshape(1, -1))
    ff = (ff_W1, ff_b1.reshape(1, -1), ff_W2, ff_b2.reshape(1, -1))
    gf = (gf_W1, gf_b1.reshape(1, -1), gf_W2, gf_b2.reshape(1, -1),
          gf_W3.reshape(1, -1), gf_b3.reshape(1, -1))
    return _final_call(hn, outf[:N_], outf[NPAD:NPAD + N_], ec[:N_],
                       Wskip, bskip.reshape(1, -1), ga,
                       ln2_w.reshape(1, -1), ln2_b.reshape(1, -1), ff, gf)


# speedup vs baseline: 3.9935x; 1.2881x over previous
import functools

import jax
import jax.numpy as jnp
import numpy as np
from jax import lax
from jax.experimental import pallas as pl
from jax.experimental.pallas import tpu as pltpu
from jax.experimental.pallas import tpu_sc as plsc

N_ = 10000
E_ = 160000
D_ = 128
H_ = 8
DE_ = 16
NPAD = 10240
EPAD = 163840
NC = 2
NS = 16
NW = NC * NS
EPW = EPAD // NW
ACCW = 144
RSQD = float(1.0 / np.sqrt(D_))
NEG = -1e9

_MESH = functools.partial(
    plsc.VectorSubcoreMesh, core_axis_name="c", subcore_axis_name="s")


def _wt_body(wq, wk, we, bq, bk,
             acat, pcat, wecat, avec, bvec, gvec, cconst):
    dn = (((1,), (1,)), ((), ()))
    for h in range(H_):
        wq_h = wq[:, h * D_:(h + 1) * D_]
        wk_h = wk[:, h * D_:(h + 1) * D_]
        we_h = we[:, h * D_:(h + 1) * D_]
        bq_h = bq[:, h * D_:(h + 1) * D_]
        bk_h = bk[:, h * D_:(h + 1) * D_]
        acat[:, h * D_:(h + 1) * D_] = lax.dot_general(
            wq_h, wk_h, dn, preferred_element_type=jnp.float32).astype(
                jnp.bfloat16)
        pcat[:, h * DE_:(h + 1) * DE_] = lax.dot_general(
            wq_h, we_h, dn, preferred_element_type=jnp.float32).astype(
                jnp.bfloat16)
        wecat[h * DE_:(h + 1) * DE_, :] = we_h
        avec[:, h:h + 1] = lax.dot_general(
            wq_h, bk_h, dn, preferred_element_type=jnp.float32)
        bvec[:, h:h + 1] = lax.dot_general(
            wk_h, bq_h, dn, preferred_element_type=jnp.float32)
        gvec[:, h:h + 1] = lax.dot_general(
            we_h, bq_h, dn, preferred_element_type=jnp.float32)
        cconst[:, h:h + 1] = jnp.sum(bq_h * bk_h, axis=1, keepdims=True)


def _wt_call(Wq, Wk, We, bq2, bk2):
    full = lambda shape: pl.BlockSpec(shape, lambda: (0, 0))
    return pl.pallas_call(
        _wt_body,
        grid=(),
        in_specs=[full((D_, H_ * D_)), full((D_, H_ * D_)), full((DE_, H_ * D_)),
                  full((1, H_ * D_)), full((1, H_ * D_))],
        out_specs=[full((D_, H_ * D_)), full((D_, H_ * DE_)), full((H_ * DE_, D_)),
                   full((D_, H_)), full((D_, H_)), full((DE_, H_)), full((1, H_))],
        out_shape=[jax.ShapeDtypeStruct((D_, H_ * D_), jnp.bfloat16),
                   jax.ShapeDtypeStruct((D_, H_ * DE_), jnp.bfloat16),
                   jax.ShapeDtypeStruct((H_ * DE_, D_), jnp.float32),
                   jax.ShapeDtypeStruct((D_, H_), jnp.float32),
                   jax.ShapeDtypeStruct((D_, H_), jnp.float32),
                   jax.ShapeDtypeStruct((DE_, H_), jnp.float32),
                   jax.ShapeDtypeStruct((1, H_), jnp.float32)],
    )(Wq, Wk, We, bq2, bk2)


def _hnv_body(h_ref, lnw, lnb, wv, bv, hn_ref, hnb_ref, v_ref):
    x = h_ref[...]
    mu = jnp.mean(x, axis=1, keepdims=True)
    var = jnp.mean((x - mu) ** 2, axis=1, keepdims=True)
    hn = (x - mu) / jnp.sqrt(var + 1e-5) * lnw[...] + lnb[...]
    hn_ref[...] = hn
    hnb_ref[...] = hn.astype(jnp.bfloat16)
    v = jnp.dot(hn, wv[...], preferred_element_type=jnp.float32) + bv[...]
    v_ref[...] = v.astype(jnp.bfloat16)


def _hnv_call(h, lnw2, lnb2, Wv, bv2):
    BN = 400
    grid = (N_ // BN,)
    row = lambda shape: pl.BlockSpec(shape, lambda i: (i, 0))
    full = lambda shape: pl.BlockSpec(shape, lambda i: (0, 0))
    return pl.pallas_call(
        _hnv_body,
        grid=grid,
        in_specs=[row((BN, D_)), full((1, D_)), full((1, D_)),
                  full((D_, H_ * D_)), full((1, H_ * D_))],
        out_specs=[row((BN, D_)), row((BN, D_)), row((BN, H_ * D_))],
        out_shape=[jax.ShapeDtypeStruct((N_, D_), jnp.float32),
                   jax.ShapeDtypeStruct((N_, D_), jnp.bfloat16),
                   jax.ShapeDtypeStruct((N_, H_ * D_), jnp.bfloat16)],
    )(h, lnw2, lnb2, Wv, bv2)


_BE = 512


def _alpha_body(hnd_ref, hns_ref, ea_ref, acat, pcat, avec, bvec, gvec, cconst,
                out_ref):
    hnd = hnd_ref[...]
    hns = hns_ref[...]
    hndf = hnd.astype(jnp.float32)
    hnsf = hns.astype(jnp.float32)
    ea = ea_ref[...]
    t1 = jnp.dot(hnd, acat[...], preferred_element_type=jnp.float32)
    hns_rep = jnp.concatenate([hnsf] * H_, axis=1)
    r1 = lax.broadcasted_iota(jnp.int32, (H_ * D_, H_), 0)
    c1 = lax.broadcasted_iota(jnp.int32, (H_ * D_, H_), 1)
    s1 = ((r1 // D_) == c1).astype(jnp.float32)
    al = jnp.dot(t1 * hns_rep, s1, preferred_element_type=jnp.float32)
    t2 = jnp.dot(hnd, pcat[...], preferred_element_type=jnp.float32)
    ea_rep = jnp.concatenate([ea] * H_, axis=1)
    r2 = lax.broadcasted_iota(jnp.int32, (H_ * DE_, H_), 0)
    c2 = lax.broadcasted_iota(jnp.int32, (H_ * DE_, H_), 1)
    s2 = ((r2 // DE_) == c2).astype(jnp.float32)
    al = al + jnp.dot(t2 * ea_rep, s2, preferred_element_type=jnp.float32)
    al = (al
          + jnp.dot(hndf, avec[...], preferred_element_type=jnp.float32)
          + jnp.dot(hnsf, bvec[...], preferred_element_type=jnp.float32)
          + jnp.dot(ea, gvec[...], preferred_element_type=jnp.float32)
          + cconst[...])
    al = al * RSQD
    al = jnp.concatenate([al, jnp.full((_BE, H_), NEG, jnp.float32)], axis=1)
    i = pl.program_id(0)
    rowid = i * _BE + lax.broadcasted_iota(jnp.int32, (_BE, 1), 0)
    out_ref[...] = jnp.where(rowid < E_, al, NEG)


def _alpha_call(hnd, hns, eap, acat, pcat, avec, bvec, gvec, cconst):
    grid = (EPAD // _BE,)
    row = lambda shape: pl.BlockSpec(shape, lambda i: (i, 0))
    full = lambda shape: pl.BlockSpec(shape, lambda i: (0, 0))
    return pl.pallas_call(
        _alpha_body,
        grid=grid,
        in_specs=[row((_BE, D_)), row((_BE, D_)), row((_BE, DE_)),
                  full((D_, H_ * D_)), full((D_, H_ * DE_)),
                  full((D_, H_)), full((D_, H_)), full((DE_, H_)),
                  full((1, H_))],
        out_specs=row((_BE, 2 * H_)),
        out_shape=jax.ShapeDtypeStruct((EPAD, 2 * H_), jnp.float32),
    )(hnd, hns, eap, acat, pcat, avec, bvec, gvec, cconst)


def _norm_body(acc0, acc1, wecat, invd_ref, ec_ref):
    den = acc0[:, D_:D_ + H_] + acc1[:, D_:D_ + H_]
    inv = 1.0 / (den + 1e-16)
    t = acc0[:, 0:D_] + acc1[:, 0:D_]
    parts = [t[:, h * DE_:(h + 1) * DE_] * inv[:, h:h + 1] for h in range(H_)]
    ts = jnp.concatenate(parts, axis=1)
    ec_ref[...] = jnp.dot(ts, wecat[...], preferred_element_type=jnp.float32)
    invd_ref[...] = jnp.concatenate([inv, jnp.zeros_like(inv)], axis=1)


def _norm_call(acc0, acc1, wecat):
    BN = 512
    grid = (NPAD // BN,)
    row = lambda shape: pl.BlockSpec(shape, lambda i: (i, 0))
    full = lambda shape: pl.BlockSpec(shape, lambda i: (0, 0))
    return pl.pallas_call(
        _norm_body,
        grid=grid,
        in_specs=[row((BN, ACCW)), row((BN, ACCW)), full((H_ * DE_, D_))],
        out_specs=[row((BN, 2 * H_)), row((BN, D_))],
        out_shape=[jax.ShapeDtypeStruct((NPAD, 2 * H_), jnp.float32),
                   jax.ShapeDtypeStruct((NPAD, D_), jnp.float32)],
    )(acc0, acc1, wecat)


def _final_body(hn_ref, o0_ref, o1_ref, ec_ref, wskip, bskip,
                gaW1, gab1, gaW2, gab2, gaW3r, gab3,
                ln2w, ln2b, ffW1, ffb1, ffW2, ffb2,
                gfW1, gfb1, gfW2, gfb2, gfW3r, gfb3, out_ref):
    hn = hn_ref[...]
    p = lax.broadcasted_iota(jnp.int32, (D_, D_), 0)
    c = lax.broadcasted_iota(jnp.int32, (D_, D_), 1)
    tgt = ((p >> 5) << 5) + 2 * (p & 15) + ((p >> 4) & 1)
    perm = (c == tgt).astype(jnp.float32)
    op = jnp.dot(o0_ref[...] + o1_ref[...], perm,
                 preferred_element_type=jnp.float32)
    out = ((op + ec_ref[...]) * (1.0 / H_)
           + jnp.dot(hn, wskip[...], preferred_element_type=jnp.float32)
           + bskip[...])

    def gate(u, v, W1, b1, W2, b2, W3r, b3):
        z = jnp.concatenate([u, v, u - v], axis=1)
        a = jnp.dot(z, W1[...], preferred_element_type=jnp.float32) + b1[...]
        a = a * jax.nn.sigmoid(a)
        a = jnp.dot(a, W2[...], preferred_element_type=jnp.float32) + b2[...]
        a = a * jax.nn.sigmoid(a)
        g = jnp.sum(a * W3r[...], axis=1, keepdims=True) + b3[...]
        g = jax.nn.sigmoid(g)
        return g * u + (1 - g) * v

    h1 = gate(hn, out, gaW1, gab1, gaW2, gab2, gaW3r, gab3)
    mu = jnp.mean(h1, axis=1, keepdims=True)
    var = jnp.mean((h1 - mu) ** 2, axis=1, keepdims=True)
    h2 = (h1 - mu) / jnp.sqrt(var + 1e-5) * ln2w[...] + ln2b[...]
    ff = jnp.dot(h2, ffW1[...], preferred_element_type=jnp.float32) + ffb1[...]
    ff = ff * jax.nn.sigmoid(ff)
    ff = jnp.dot(ff, ffW2[...], preferred_element_type=jnp.float32) + ffb2[...]
    out_ref[...] = gate(h2, ff, gfW1, gfb1, gfW2, gfb2, gfW3r, gfb3)


def _final_call(hn, o0, o1, ec, Wskip, bskip2, ga, ln2w2, ln2b2, ff, gf):
    BN = 400
    grid = (N_ // BN,)
    row = lambda shape: pl.BlockSpec(shape, lambda i: (i, 0))
    full = lambda shape: pl.BlockSpec(shape, lambda i: (0, 0))
    D3, D32, D34 = 3 * D_, 3 * D_ // 2, 3 * D_ // 4
    in_specs = [row((BN, D_)), row((BN, D_)), row((BN, D_)), row((BN, D_)),
                full((D_, D_)), full((1, D_)),
                full((D3, D32)), full((1, D32)), full((D32, D34)), full((1, D34)),
                full((1, D34)), full((1, 1)),
                full((1, D_)), full((1, D_)),
                full((D_, D_)), full((1, D_)), full((D_, D_)), full((1, D_)),
                full((D3, D32)), full((1, D32)), full((D32, D34)), full((1, D34)),
                full((1, D34)), full((1, 1))]
    return pl.pallas_call(
        _final_body,
        grid=grid,
        in_specs=in_specs,
        out_specs=row((BN, D_)),
        out_shape=jax.ShapeDtypeStruct((N_, D_), jnp.float32),
    )(hn, o0, o1, ec, Wskip, bskip2, *ga, ln2w2, ln2b2, *ff, *gf)


def _sc_gather(hnb, srcp, dstp):
    CB = 128
    nch = EPW // CB

    @functools.partial(
        pl.kernel,
        out_type=(jax.ShapeDtypeStruct((EPAD, D_), jnp.bfloat16),
                  jax.ShapeDtypeStruct((EPAD, D_), jnp.bfloat16)),
        mesh=_MESH(),
        compiler_params=pltpu.CompilerParams(use_tc_tiling_on_sc=False, needs_layout_passes=False),
        scratch_types=[pltpu.VMEM((EPW // 128, 128), jnp.int32),
                       pltpu.VMEM((EPW // 128, 128), jnp.int32),
                       pltpu.VMEM((2, CB, D_), jnp.bfloat16),
                       pltpu.VMEM((2, CB, D_), jnp.bfloat16),
                       pltpu.SemaphoreType.DMA, pltpu.SemaphoreType.DMA,
                       pltpu.SemaphoreType.DMA, pltpu.SemaphoreType.DMA],
    )
    def k(hn_hbm, src_hbm, dst_hbm, hns_hbm, hnd_hbm,
          sidx, didx, srows, drows, s_s0, s_s1, s_d0, s_d1):
        wid = lax.axis_index("s") * NC + lax.axis_index("c")
        ssems = (s_s0, s_s1)
        dsems = (s_d0, s_d1)

        pltpu.sync_copy(src_hbm.at[pl.ds(wid * nch, nch)], sidx)
        pltpu.sync_copy(dst_hbm.at[pl.ds(wid * nch, nch)], didx)

        def start(j, b):
            pltpu.async_copy(hn_hbm.at[sidx.at[j]], srows.at[b], ssems[b])
            pltpu.async_copy(hn_hbm.at[didx.at[j]], drows.at[b], dsems[b])

        def drain(j, b):
            base = wid * EPW + j * CB
            pltpu.make_async_copy(hn_hbm.at[sidx.at[j]], srows.at[b],
                                  ssems[b]).wait()
            pltpu.make_async_copy(hn_hbm.at[didx.at[j]], drows.at[b],
                                  dsems[b]).wait()
            pltpu.sync_copy(srows.at[b], hns_hbm.at[pl.ds(base, CB)])
            pltpu.sync_copy(drows.at[b], hnd_hbm.at[pl.ds(base, CB)])

        start(0, 0)

        def body(p, carry):
            start(2 * p + 1, 1)
            drain(2 * p, 0)

            @pl.when(p + 1 < nch // 2)
            def _():
                start(2 * p + 2, 0)
            drain(2 * p + 1, 1)
            return carry

        lax.fori_loop(0, nch // 2, body, 0)

    return k(hnb, srcp.reshape(EPAD // CB, CB), dstp.reshape(EPAD // CB, CB))


def _sc_p1(alpha, eap, dstp):
    CB = 128
    nch = EPW // CB
    rpt = NPAD // NS

    @functools.partial(
        pl.kernel,
        out_type=(jax.ShapeDtypeStruct((NPAD, ACCW), jnp.float32),
                  jax.ShapeDtypeStruct((NPAD, ACCW), jnp.float32)),
        mesh=_MESH(),
        compiler_params=pltpu.CompilerParams(use_tc_tiling_on_sc=False, needs_layout_passes=False),
        scratch_types=[pltpu.VMEM((CB, 2 * H_), jnp.float32),
                       pltpu.VMEM((CB, DE_), jnp.float32),
                       pltpu.VMEM((nch, CB), jnp.int32),
                       pltpu.VMEM((CB, ACCW), jnp.float32),
                       pltpu.VMEM_SHARED((NPAD, ACCW), jnp.float32)],
    )
    def k(alpha_hbm, ea_hbm, dst_hbm, out0_hbm, out1_hbm,
          abuf, eabuf, didx, payload, acc):
        cid = lax.axis_index("c")
        sid = lax.axis_index("s")
        wid = sid * NC + cid

        pltpu.sync_copy(dst_hbm.at[pl.ds(wid * nch, nch)], didx)

        def zrow(i, c):
            for j in range(ACCW // 16):
                payload[i, pl.ds(j * 16, 16)] = jnp.zeros((16,), jnp.float32)
            return c
        lax.fori_loop(0, CB, zrow, 0)

        def zacc(i, c):
            pltpu.sync_copy(payload, acc.at[pl.ds(sid * rpt + i * CB, CB)])
            return c
        lax.fori_loop(0, rpt // CB, zacc, 0)
        plsc.subcore_barrier()

        def chunk(j, carry):
            base = wid * EPW + j * CB
            pltpu.sync_copy(alpha_hbm.at[pl.ds(base, CB)], abuf)
            pltpu.sync_copy(ea_hbm.at[pl.ds(base, CB)], eabuf)

            def edge(e, c2):
                ex16 = jnp.exp(abuf[e, pl.ds(0, 16)])
                payload[e, pl.ds(D_, 16)] = ex16
                earow = eabuf[e, pl.ds(0, DE_)]
                for h in range(H_):
                    payload[e, pl.ds(h * DE_, DE_)] = (
                        jnp.full((16,), ex16[h]) * earow)
                return c2

            lax.fori_loop(0, CB, edge, 0)
            pltpu.sync_copy(payload, acc.at[didx.at[j]], add=True)
            return carry

        lax.fori_loop(0, nch, chunk, 0)
        plsc.subcore_barrier()

        def wout(i, c):
            r0 = sid * rpt + i * CB

            @pl.when(cid == 0)
            def _():
                pltpu.sync_copy(acc.at[pl.ds(r0, CB)],
                                out0_hbm.at[pl.ds(r0, CB)])

            @pl.when(cid == 1)
            def _():
                pltpu.sync_copy(acc.at[pl.ds(r0, CB)],
                                out1_hbm.at[pl.ds(r0, CB)])
            return c
        lax.fori_loop(0, rpt // CB, wout, 0)

    return k(alpha, eap, dstp.reshape(EPAD // CB, CB))


def _sc_p2(alpha, srcp, dstp, Vb, invd):
    CB = 32
    nch = EPW // CB
    rpt = NPAD // NS

    @functools.partial(
        pl.kernel,
        out_type=(jax.ShapeDtypeStruct((NPAD, D_), jnp.float32),
                  jax.ShapeDtypeStruct((NPAD, D_), jnp.float32)),
        mesh=_MESH(),
        compiler_params=pltpu.CompilerParams(use_tc_tiling_on_sc=False, needs_layout_passes=False),
        scratch_types=[pltpu.VMEM((2, CB, 2 * H_), jnp.float32),
                       pltpu.VMEM((2, CB, 2 * H_), jnp.float32),
                       pltpu.VMEM((nch, CB), jnp.int32),
                       pltpu.VMEM((nch, CB), jnp.int32),
                       pltpu.VMEM((2, CB, H_ * D_), jnp.bfloat16),
                       pltpu.VMEM((CB, D_), jnp.float32),
                       pltpu.VMEM_SHARED((NPAD, D_), jnp.float32),
                       pltpu.SemaphoreType.DMA, pltpu.SemaphoreType.DMA,
                       pltpu.SemaphoreType.DMA, pltpu.SemaphoreType.DMA],
    )
    def k(alpha_hbm, src_hbm, dst_hbm, v_hbm, invd_hbm, out0_hbm, out1_hbm,
          abuf, ivbuf, sidx, didx, vrows, wpay, acc, sv0, sv1, si0, si1):
        cid = lax.axis_index("c")
        sid = lax.axis_index("s")
        wid = sid * NC + cid
        vsems = (sv0, sv1)
        isems = (si0, si1)

        pltpu.sync_copy(src_hbm.at[pl.ds(wid * nch, nch)], sidx)
        pltpu.sync_copy(dst_hbm.at[pl.ds(wid * nch, nch)], didx)

        def zrow(i, c):
            for j in range(D_ // 16):
                wpay[i, pl.ds(j * 16, 16)] = jnp.zeros((16,), jnp.float32)
            return c
        lax.fori_loop(0, CB, zrow, 0)

        def zacc(i, c):
            pltpu.sync_copy(wpay, acc.at[pl.ds(sid * rpt + i * CB, CB)])
            return c
        lax.fori_loop(0, rpt // CB, zacc, 0)
        plsc.subcore_barrier()

        def start(j, b):
            base = wid * EPW + j * CB
            pltpu.sync_copy(alpha_hbm.at[pl.ds(base, CB)], abuf.at[b])
            pltpu.async_copy(v_hbm.at[sidx.at[j]], vrows.at[b], vsems[b])
            pltpu.async_copy(invd_hbm.at[didx.at[j]], ivbuf.at[b], isems[b])

        def process(j, b):
            pltpu.make_async_copy(
                v_hbm.at[sidx.at[j]], vrows.at[b], vsems[b]).wait()
            pltpu.make_async_copy(
                invd_hbm.at[didx.at[j]], ivbuf.at[b], isems[b]).wait()

            def edge(e, c2):
                attn16 = (jnp.exp(abuf[b, e, pl.ds(0, 16)])
                          * ivbuf[b, e, pl.ds(0, 16)])
                accs = [jnp.zeros((16,), jnp.float32)
                        for _ in range(D_ // 16)]
                for h in range(H_):
                    avv = jnp.full((16,), attn16[h])
                    for g in range(D_ // 32):
                        x32 = vrows[b, e, pl.ds(h * D_ + g * 32, 32)]
                        lo, hi = plsc.unpack(
                            x32, format=plsc.PackFormat.INTERLEAVED)
                        accs[2 * g] = accs[2 * g] + avv * lo
                        accs[2 * g + 1] = accs[2 * g + 1] + avv * hi
                for dv in range(D_ // 16):
                    wpay[e, pl.ds(dv * 16, 16)] = accs[dv]
                return c2

            lax.fori_loop(0, CB, edge, 0)
            pltpu.sync_copy(wpay, acc.at[didx.at[j]], add=True)

        start(0, 0)

        def body(p, carry):
            start(2 * p + 1, 1)
            process(2 * p, 0)

            @pl.when(p + 1 < nch // 2)
            def _():
                start(2 * p + 2, 0)
            process(2 * p + 1, 1)
            return carry

        lax.fori_loop(0, nch // 2, body, 0)
        plsc.subcore_barrier()

        def wout(i, c):
            r0 = sid * rpt + i * CB

            @pl.when(cid == 0)
            def _():
                pltpu.sync_copy(acc.at[pl.ds(r0, CB)],
                                out0_hbm.at[pl.ds(r0, CB)])

            @pl.when(cid == 1)
            def _():
                pltpu.sync_copy(acc.at[pl.ds(r0, CB)],
                                out1_hbm.at[pl.ds(r0, CB)])
            return c
        lax.fori_loop(0, rpt // CB, wout, 0)

    return k(alpha, srcp.reshape(EPAD // CB, CB), dstp.reshape(EPAD // CB, CB),
             Vb, invd)


def kernel(h, edge_index, edge_attr, ln1_w, ln1_b, Wq, bq, Wk, bk, Wv, bv, We,
           Wskip, bskip, ga_W1, ga_b1, ga_W2, ga_b2, ga_W3, ga_b3, ln2_w, ln2_b,
           ff_W1, ff_b1, ff_W2, ff_b2, gf_W1, gf_b1, gf_W2, gf_b2, gf_W3, gf_b3):
    pad_e = EPAD - E_
    srcp = jnp.concatenate([edge_index[0], jnp.zeros((pad_e,), jnp.int32)])
    dstp = jnp.concatenate([edge_index[1], jnp.zeros((pad_e,), jnp.int32)])
    eap = jnp.concatenate(
        [edge_attr, jnp.zeros((pad_e, DE_), jnp.float32)], axis=0)

    acat, pcat, wecat, avec, bvec, gvec, cconst = _wt_call(
        Wq, Wk, We, bq.reshape(1, -1), bk.reshape(1, -1))
    hn, hnb, Vb = _hnv_call(h, ln1_w.reshape(1, -1), ln1_b.reshape(1, -1),
                            Wv, bv.reshape(1, -1))
    hns, hnd = _sc_gather(hnb, srcp, dstp)
    alpha = _alpha_call(hnd, hns, eap, acat, pcat, avec, bvec, gvec, cconst)
    acc0, acc1 = _sc_p1(alpha, eap, dstp)
    invd, ec = _norm_call(acc0, acc1, wecat)
    o0, o1 = _sc_p2(alpha, srcp, dstp, Vb, invd)
    ga = (ga_W1, ga_b1.reshape(1, -1), ga_W2, ga_b2.reshape(1, -1),
          ga_W3.reshape(1, -1), ga_b3.reshape(1, -1))
    ff = (ff_W1, ff_b1.reshape(1, -1), ff_W2, ff_b2.reshape(1, -1))
    gf = (gf_W1, gf_b1.reshape(1, -1), gf_W2, gf_b2.reshape(1, -1),
          gf_W3.reshape(1, -1), gf_b3.reshape(1, -1))
    return _final_call(hn, o0, o1, ec,
                       Wskip, bskip.reshape(1, -1), ga,
                       ln2_w.reshape(1, -1), ln2_b.reshape(1, -1), ff, gf)


# speedup vs baseline: 4.1010x; 1.0269x over previous
import functools

import jax
import jax.numpy as jnp
import numpy as np
from jax import lax
from jax.experimental import pallas as pl
from jax.experimental.pallas import tpu as pltpu
from jax.experimental.pallas import tpu_sc as plsc

N_ = 10000
E_ = 160000
D_ = 128
H_ = 8
DE_ = 16
NPAD = 10240
EPAD = 163840
NC = 2
NS = 16
NW = NC * NS
EPW = EPAD // NW
ACCW = 144
RSQD = float(1.0 / np.sqrt(D_))
NEG = -1e9

_MESH = functools.partial(
    plsc.VectorSubcoreMesh, core_axis_name="c", subcore_axis_name="s")


def _wt_body(wq, wk, we, bq, bk,
             acat, pcat, wecat, avec, bvec, gvec, cconst):
    dn = (((1,), (1,)), ((), ()))
    for h in range(H_):
        wq_h = wq[:, h * D_:(h + 1) * D_]
        wk_h = wk[:, h * D_:(h + 1) * D_]
        we_h = we[:, h * D_:(h + 1) * D_]
        bq_h = bq[:, h * D_:(h + 1) * D_]
        bk_h = bk[:, h * D_:(h + 1) * D_]
        acat[:, h * D_:(h + 1) * D_] = lax.dot_general(
            wq_h, wk_h, dn, preferred_element_type=jnp.float32).astype(
                jnp.bfloat16)
        pcat[:, h * DE_:(h + 1) * DE_] = lax.dot_general(
            wq_h, we_h, dn, preferred_element_type=jnp.float32).astype(
                jnp.bfloat16)
        wecat[h * DE_:(h + 1) * DE_, :] = we_h
        avec[:, h:h + 1] = lax.dot_general(
            wq_h, bk_h, dn, preferred_element_type=jnp.float32)
        bvec[:, h:h + 1] = lax.dot_general(
            wk_h, bq_h, dn, preferred_element_type=jnp.float32)
        gvec[:, h:h + 1] = lax.dot_general(
            we_h, bq_h, dn, preferred_element_type=jnp.float32)
        cconst[:, h:h + 1] = jnp.sum(bq_h * bk_h, axis=1, keepdims=True)


def _wt_call(Wq, Wk, We, bq2, bk2):
    full = lambda shape: pl.BlockSpec(shape, lambda: (0, 0))
    return pl.pallas_call(
        _wt_body,
        grid=(),
        in_specs=[full((D_, H_ * D_)), full((D_, H_ * D_)), full((DE_, H_ * D_)),
                  full((1, H_ * D_)), full((1, H_ * D_))],
        out_specs=[full((D_, H_ * D_)), full((D_, H_ * DE_)), full((H_ * DE_, D_)),
                   full((D_, H_)), full((D_, H_)), full((DE_, H_)), full((1, H_))],
        out_shape=[jax.ShapeDtypeStruct((D_, H_ * D_), jnp.bfloat16),
                   jax.ShapeDtypeStruct((D_, H_ * DE_), jnp.bfloat16),
                   jax.ShapeDtypeStruct((H_ * DE_, D_), jnp.float32),
                   jax.ShapeDtypeStruct((D_, H_), jnp.float32),
                   jax.ShapeDtypeStruct((D_, H_), jnp.float32),
                   jax.ShapeDtypeStruct((DE_, H_), jnp.float32),
                   jax.ShapeDtypeStruct((1, H_), jnp.float32)],
    )(Wq, Wk, We, bq2, bk2)


def _hnv_body(h_ref, lnw, lnb, wv, bv, hn_ref, hnb_ref, v_ref):
    x = h_ref[...]
    mu = jnp.mean(x, axis=1, keepdims=True)
    var = jnp.mean((x - mu) ** 2, axis=1, keepdims=True)
    hn = (x - mu) / jnp.sqrt(var + 1e-5) * lnw[...] + lnb[...]
    hn_ref[...] = hn
    hnb_ref[...] = hn.astype(jnp.bfloat16)
    v = jnp.dot(hn, wv[...], preferred_element_type=jnp.float32) + bv[...]
    v_ref[...] = v.astype(jnp.bfloat16)


def _hnv_call(h, lnw2, lnb2, Wv, bv2):
    BN = 400
    grid = (N_ // BN,)
    row = lambda shape: pl.BlockSpec(shape, lambda i: (i, 0))
    full = lambda shape: pl.BlockSpec(shape, lambda i: (0, 0))
    return pl.pallas_call(
        _hnv_body,
        grid=grid,
        in_specs=[row((BN, D_)), full((1, D_)), full((1, D_)),
                  full((D_, H_ * D_)), full((1, H_ * D_))],
        out_specs=[row((BN, D_)), row((BN, D_)), row((BN, H_ * D_))],
        out_shape=[jax.ShapeDtypeStruct((N_, D_), jnp.float32),
                   jax.ShapeDtypeStruct((N_, D_), jnp.bfloat16),
                   jax.ShapeDtypeStruct((N_, H_ * D_), jnp.bfloat16)],
    )(h, lnw2, lnb2, Wv, bv2)


_BE = 512


def _alpha_body(hnd_ref, hns_ref, ea_ref, acat, pcat, avec, bvec, gvec, cconst,
                out_ref):
    hnd = hnd_ref[...]
    hns = hns_ref[...]
    hndf = hnd.astype(jnp.float32)
    hnsf = hns.astype(jnp.float32)
    ea = ea_ref[...]
    t1 = jnp.dot(hnd, acat[...], preferred_element_type=jnp.float32)
    hns_rep = jnp.concatenate([hnsf] * H_, axis=1)
    r1 = lax.broadcasted_iota(jnp.int32, (H_ * D_, H_), 0)
    c1 = lax.broadcasted_iota(jnp.int32, (H_ * D_, H_), 1)
    s1 = ((r1 // D_) == c1).astype(jnp.float32)
    al = jnp.dot(t1 * hns_rep, s1, preferred_element_type=jnp.float32)
    t2 = jnp.dot(hnd, pcat[...], preferred_element_type=jnp.float32)
    ea_rep = jnp.concatenate([ea] * H_, axis=1)
    r2 = lax.broadcasted_iota(jnp.int32, (H_ * DE_, H_), 0)
    c2 = lax.broadcasted_iota(jnp.int32, (H_ * DE_, H_), 1)
    s2 = ((r2 // DE_) == c2).astype(jnp.float32)
    al = al + jnp.dot(t2 * ea_rep, s2, preferred_element_type=jnp.float32)
    al = (al
          + jnp.dot(hndf, avec[...], preferred_element_type=jnp.float32)
          + jnp.dot(hnsf, bvec[...], preferred_element_type=jnp.float32)
          + jnp.dot(ea, gvec[...], preferred_element_type=jnp.float32)
          + cconst[...])
    al = al * RSQD
    al = jnp.concatenate([al, jnp.full((_BE, H_), NEG, jnp.float32)], axis=1)
    i = pl.program_id(0)
    rowid = i * _BE + lax.broadcasted_iota(jnp.int32, (_BE, 1), 0)
    out_ref[...] = jnp.where(rowid < E_, al, NEG)


def _alpha_call(hnd, hns, eap, acat, pcat, avec, bvec, gvec, cconst):
    grid = (EPAD // _BE,)
    row = lambda shape: pl.BlockSpec(shape, lambda i: (i, 0))
    full = lambda shape: pl.BlockSpec(shape, lambda i: (0, 0))
    return pl.pallas_call(
        _alpha_body,
        grid=grid,
        in_specs=[row((_BE, D_)), row((_BE, D_)), row((_BE, DE_)),
                  full((D_, H_ * D_)), full((D_, H_ * DE_)),
                  full((D_, H_)), full((D_, H_)), full((DE_, H_)),
                  full((1, H_))],
        out_specs=row((_BE, 2 * H_)),
        out_shape=jax.ShapeDtypeStruct((EPAD, 2 * H_), jnp.float32),
    )(hnd, hns, eap, acat, pcat, avec, bvec, gvec, cconst)


def _norm_body(acc0, acc1, wecat, invd_ref, ec_ref):
    den = acc0[:, D_:D_ + H_] + acc1[:, D_:D_ + H_]
    inv = 1.0 / (den + 1e-16)
    t = acc0[:, 0:D_] + acc1[:, 0:D_]
    parts = [t[:, h * DE_:(h + 1) * DE_] * inv[:, h:h + 1] for h in range(H_)]
    ts = jnp.concatenate(parts, axis=1)
    ec_ref[...] = jnp.dot(ts, wecat[...], preferred_element_type=jnp.float32)
    invd_ref[...] = jnp.concatenate([inv, jnp.zeros_like(inv)], axis=1)


def _norm_call(acc0, acc1, wecat):
    BN = 512
    grid = (NPAD // BN,)
    row = lambda shape: pl.BlockSpec(shape, lambda i: (i, 0))
    full = lambda shape: pl.BlockSpec(shape, lambda i: (0, 0))
    return pl.pallas_call(
        _norm_body,
        grid=grid,
        in_specs=[row((BN, ACCW)), row((BN, ACCW)), full((H_ * DE_, D_))],
        out_specs=[row((BN, 2 * H_)), row((BN, D_))],
        out_shape=[jax.ShapeDtypeStruct((NPAD, 2 * H_), jnp.float32),
                   jax.ShapeDtypeStruct((NPAD, D_), jnp.float32)],
    )(acc0, acc1, wecat)


def _final_body(hn_ref, o0_ref, o1_ref, ec_ref, wskip, bskip,
                gaW1, gab1, gaW2, gab2, gaW3r, gab3,
                ln2w, ln2b, ffW1, ffb1, ffW2, ffb2,
                gfW1, gfb1, gfW2, gfb2, gfW3r, gfb3, out_ref):
    hn = hn_ref[...]
    p = lax.broadcasted_iota(jnp.int32, (D_, D_), 0)
    c = lax.broadcasted_iota(jnp.int32, (D_, D_), 1)
    tgt = ((p >> 5) << 5) + 2 * (p & 15) + ((p >> 4) & 1)
    perm = (c == tgt).astype(jnp.float32)
    op = jnp.dot(o0_ref[...] + o1_ref[...], perm,
                 preferred_element_type=jnp.float32)
    out = ((op + ec_ref[...]) * (1.0 / H_)
           + jnp.dot(hn, wskip[...], preferred_element_type=jnp.float32)
           + bskip[...])

    def gate(u, v, W1, b1, W2, b2, W3r, b3):
        z = jnp.concatenate([u, v, u - v], axis=1)
        a = jnp.dot(z, W1[...], preferred_element_type=jnp.float32) + b1[...]
        a = a * jax.nn.sigmoid(a)
        a = jnp.dot(a, W2[...], preferred_element_type=jnp.float32) + b2[...]
        a = a * jax.nn.sigmoid(a)
        g = jnp.sum(a * W3r[...], axis=1, keepdims=True) + b3[...]
        g = jax.nn.sigmoid(g)
        return g * u + (1 - g) * v

    h1 = gate(hn, out, gaW1, gab1, gaW2, gab2, gaW3r, gab3)
    mu = jnp.mean(h1, axis=1, keepdims=True)
    var = jnp.mean((h1 - mu) ** 2, axis=1, keepdims=True)
    h2 = (h1 - mu) / jnp.sqrt(var + 1e-5) * ln2w[...] + ln2b[...]
    ff = jnp.dot(h2, ffW1[...], preferred_element_type=jnp.float32) + ffb1[...]
    ff = ff * jax.nn.sigmoid(ff)
    ff = jnp.dot(ff, ffW2[...], preferred_element_type=jnp.float32) + ffb2[...]
    out_ref[...] = gate(h2, ff, gfW1, gfb1, gfW2, gfb2, gfW3r, gfb3)


def _final_call(hn, o0, o1, ec, Wskip, bskip2, ga, ln2w2, ln2b2, ff, gf):
    BN = 400
    grid = (N_ // BN,)
    row = lambda shape: pl.BlockSpec(shape, lambda i: (i, 0))
    full = lambda shape: pl.BlockSpec(shape, lambda i: (0, 0))
    D3, D32, D34 = 3 * D_, 3 * D_ // 2, 3 * D_ // 4
    in_specs = [row((BN, D_)), row((BN, D_)), row((BN, D_)), row((BN, D_)),
                full((D_, D_)), full((1, D_)),
                full((D3, D32)), full((1, D32)), full((D32, D34)), full((1, D34)),
                full((1, D34)), full((1, 1)),
                full((1, D_)), full((1, D_)),
                full((D_, D_)), full((1, D_)), full((D_, D_)), full((1, D_)),
                full((D3, D32)), full((1, D32)), full((D32, D34)), full((1, D34)),
                full((1, D34)), full((1, 1))]
    return pl.pallas_call(
        _final_body,
        grid=grid,
        in_specs=in_specs,
        out_specs=row((BN, D_)),
        out_shape=jax.ShapeDtypeStruct((N_, D_), jnp.float32),
    )(hn, o0, o1, ec, Wskip, bskip2, *ga, ln2w2, ln2b2, *ff, *gf)


def _sc_gather(hnb, srcp, dstp):
    CB = 128
    nch = EPW // CB

    @functools.partial(
        pl.kernel,
        out_type=(jax.ShapeDtypeStruct((EPAD, D_), jnp.bfloat16),
                  jax.ShapeDtypeStruct((EPAD, D_), jnp.bfloat16)),
        mesh=_MESH(),
        compiler_params=pltpu.CompilerParams(use_tc_tiling_on_sc=False, needs_layout_passes=False),
        scratch_types=[pltpu.VMEM((EPW // 128, 128), jnp.int32),
                       pltpu.VMEM((EPW // 128, 128), jnp.int32),
                       pltpu.VMEM((2, CB, D_), jnp.bfloat16),
                       pltpu.VMEM((2, CB, D_), jnp.bfloat16),
                       pltpu.SemaphoreType.DMA, pltpu.SemaphoreType.DMA,
                       pltpu.SemaphoreType.DMA, pltpu.SemaphoreType.DMA],
    )
    def k(hn_hbm, src_hbm, dst_hbm, hns_hbm, hnd_hbm,
          sidx, didx, srows, drows, s_s0, s_s1, s_d0, s_d1):
        wid = lax.axis_index("s") * NC + lax.axis_index("c")
        ssems = (s_s0, s_s1)
        dsems = (s_d0, s_d1)

        pltpu.sync_copy(src_hbm.at[pl.ds(wid * nch, nch)], sidx)
        pltpu.sync_copy(dst_hbm.at[pl.ds(wid * nch, nch)], didx)

        def start(j, b):
            pltpu.async_copy(hn_hbm.at[sidx.at[j]], srows.at[b], ssems[b])
            pltpu.async_copy(hn_hbm.at[didx.at[j]], drows.at[b], dsems[b])

        def drain(j, b):
            base = wid * EPW + j * CB
            pltpu.make_async_copy(hn_hbm.at[sidx.at[j]], srows.at[b],
                                  ssems[b]).wait()
            pltpu.make_async_copy(hn_hbm.at[didx.at[j]], drows.at[b],
                                  dsems[b]).wait()
            pltpu.sync_copy(srows.at[b], hns_hbm.at[pl.ds(base, CB)])
            pltpu.sync_copy(drows.at[b], hnd_hbm.at[pl.ds(base, CB)])

        start(0, 0)

        def body(p, carry):
            start(2 * p + 1, 1)
            drain(2 * p, 0)

            @pl.when(p + 1 < nch // 2)
            def _():
                start(2 * p + 2, 0)
            drain(2 * p + 1, 1)
            return carry

        lax.fori_loop(0, nch // 2, body, 0)

    return k(hnb, srcp.reshape(EPAD // CB, CB), dstp.reshape(EPAD // CB, CB))


def _sc_p1(alpha, eap, dstp):
    CB = 128
    nch = EPW // CB
    rpt = NPAD // NS

    @functools.partial(
        pl.kernel,
        out_type=(jax.ShapeDtypeStruct((NPAD, ACCW), jnp.float32),
                  jax.ShapeDtypeStruct((NPAD, ACCW), jnp.float32)),
        mesh=_MESH(),
        compiler_params=pltpu.CompilerParams(use_tc_tiling_on_sc=False, needs_layout_passes=False),
        scratch_types=[pltpu.VMEM((2, CB, 2 * H_), jnp.float32),
                       pltpu.VMEM((2, CB, DE_), jnp.float32),
                       pltpu.VMEM((nch, CB), jnp.int32),
                       pltpu.VMEM((CB, ACCW), jnp.float32),
                       pltpu.VMEM_SHARED((NPAD, ACCW), jnp.float32),
                       pltpu.SemaphoreType.DMA, pltpu.SemaphoreType.DMA,
                       pltpu.SemaphoreType.DMA, pltpu.SemaphoreType.DMA],
    )
    def k(alpha_hbm, ea_hbm, dst_hbm, out0_hbm, out1_hbm,
          abuf, eabuf, didx, payload, acc, sa0, sa1, se0, se1):
        cid = lax.axis_index("c")
        sid = lax.axis_index("s")
        wid = sid * NC + cid
        asems = (sa0, sa1)
        esems = (se0, se1)

        pltpu.sync_copy(dst_hbm.at[pl.ds(wid * nch, nch)], didx)

        def zrow(i, c):
            for j in range(ACCW // 16):
                payload[i, pl.ds(j * 16, 16)] = jnp.zeros((16,), jnp.float32)
            return c
        lax.fori_loop(0, CB, zrow, 0)

        def zacc(i, c):
            pltpu.sync_copy(payload, acc.at[pl.ds(sid * rpt + i * CB, CB)])
            return c
        lax.fori_loop(0, rpt // CB, zacc, 0)
        plsc.subcore_barrier()

        def start(j, b):
            base = wid * EPW + j * CB
            pltpu.async_copy(alpha_hbm.at[pl.ds(base, CB)], abuf.at[b],
                             asems[b])
            pltpu.async_copy(ea_hbm.at[pl.ds(base, CB)], eabuf.at[b],
                             esems[b])

        def process(j, b):
            base = wid * EPW + j * CB
            pltpu.make_async_copy(alpha_hbm.at[pl.ds(base, CB)], abuf.at[b],
                                  asems[b]).wait()
            pltpu.make_async_copy(ea_hbm.at[pl.ds(base, CB)], eabuf.at[b],
                                  esems[b]).wait()

            def edge(e, c2):
                ex16 = jnp.exp(abuf[b, e, pl.ds(0, 16)])
                payload[e, pl.ds(D_, 16)] = ex16
                earow = eabuf[b, e, pl.ds(0, DE_)]
                for h in range(H_):
                    payload[e, pl.ds(h * DE_, DE_)] = (
                        jnp.full((16,), ex16[h]) * earow)
                return c2

            lax.fori_loop(0, CB, edge, 0)
            pltpu.sync_copy(payload, acc.at[didx.at[j]], add=True)

        start(0, 0)

        def body(p, carry):
            start(2 * p + 1, 1)
            process(2 * p, 0)

            @pl.when(p + 1 < nch // 2)
            def _():
                start(2 * p + 2, 0)
            process(2 * p + 1, 1)
            return carry

        lax.fori_loop(0, nch // 2, body, 0)
        plsc.subcore_barrier()

        def wout(i, c):
            r0 = sid * rpt + i * CB

            @pl.when(cid == 0)
            def _():
                pltpu.sync_copy(acc.at[pl.ds(r0, CB)],
                                out0_hbm.at[pl.ds(r0, CB)])

            @pl.when(cid == 1)
            def _():
                pltpu.sync_copy(acc.at[pl.ds(r0, CB)],
                                out1_hbm.at[pl.ds(r0, CB)])
            return c
        lax.fori_loop(0, rpt // CB, wout, 0)

    return k(alpha, eap, dstp.reshape(EPAD // CB, CB))


def _sc_p2(alpha, srcp, dstp, Vb, invd):
    CB = 32
    nch = EPW // CB
    rpt = NPAD // NS

    @functools.partial(
        pl.kernel,
        out_type=(jax.ShapeDtypeStruct((NPAD, D_), jnp.float32),
                  jax.ShapeDtypeStruct((NPAD, D_), jnp.float32)),
        mesh=_MESH(),
        compiler_params=pltpu.CompilerParams(use_tc_tiling_on_sc=False, needs_layout_passes=False),
        scratch_types=[pltpu.VMEM((2, CB, 2 * H_), jnp.float32),
                       pltpu.VMEM((2, CB, 2 * H_), jnp.float32),
                       pltpu.VMEM((nch, CB), jnp.int32),
                       pltpu.VMEM((nch, CB), jnp.int32),
                       pltpu.VMEM((2, CB, H_ * D_), jnp.bfloat16),
                       pltpu.VMEM((CB, D_), jnp.float32),
                       pltpu.VMEM_SHARED((NPAD, D_), jnp.float32),
                       pltpu.SemaphoreType.DMA, pltpu.SemaphoreType.DMA,
                       pltpu.SemaphoreType.DMA, pltpu.SemaphoreType.DMA],
    )
    def k(alpha_hbm, src_hbm, dst_hbm, v_hbm, invd_hbm, out0_hbm, out1_hbm,
          abuf, ivbuf, sidx, didx, vrows, wpay, acc, sv0, sv1, si0, si1):
        cid = lax.axis_index("c")
        sid = lax.axis_index("s")
        wid = sid * NC + cid
        vsems = (sv0, sv1)
        isems = (si0, si1)

        pltpu.sync_copy(src_hbm.at[pl.ds(wid * nch, nch)], sidx)
        pltpu.sync_copy(dst_hbm.at[pl.ds(wid * nch, nch)], didx)

        def zrow(i, c):
            for j in range(D_ // 16):
                wpay[i, pl.ds(j * 16, 16)] = jnp.zeros((16,), jnp.float32)
            return c
        lax.fori_loop(0, CB, zrow, 0)

        def zacc(i, c):
            pltpu.sync_copy(wpay, acc.at[pl.ds(sid * rpt + i * CB, CB)])
            return c
        lax.fori_loop(0, rpt // CB, zacc, 0)
        plsc.subcore_barrier()

        def start(j, b):
            base = wid * EPW + j * CB
            pltpu.sync_copy(alpha_hbm.at[pl.ds(base, CB)], abuf.at[b])
            pltpu.async_copy(v_hbm.at[sidx.at[j]], vrows.at[b], vsems[b])
            pltpu.async_copy(invd_hbm.at[didx.at[j]], ivbuf.at[b], isems[b])

        def process(j, b):
            pltpu.make_async_copy(
                v_hbm.at[sidx.at[j]], vrows.at[b], vsems[b]).wait()
            pltpu.make_async_copy(
                invd_hbm.at[didx.at[j]], ivbuf.at[b], isems[b]).wait()

            def edge(e, c2):
                attn16 = (jnp.exp(abuf[b, e, pl.ds(0, 16)])
                          * ivbuf[b, e, pl.ds(0, 16)])
                accs = [jnp.zeros((16,), jnp.float32)
                        for _ in range(D_ // 16)]
                for h in range(H_):
                    avf = jnp.full((16,), attn16[h])
                    avv = plsc.pack(avf, avf,
                                    format=plsc.PackFormat.INTERLEAVED)
                    for g in range(D_ // 32):
                        x32 = vrows[b, e, pl.ds(h * D_ + g * 32, 32)]
                        lo, hi = plsc.unpack(
                            x32 * avv, format=plsc.PackFormat.INTERLEAVED)
                        accs[2 * g] = accs[2 * g] + lo
                        accs[2 * g + 1] = accs[2 * g + 1] + hi
                for dv in range(D_ // 16):
                    wpay[e, pl.ds(dv * 16, 16)] = accs[dv]
                return c2

            lax.fori_loop(0, CB, edge, 0)
            pltpu.sync_copy(wpay, acc.at[didx.at[j]], add=True)

        start(0, 0)

        def body(p, carry):
            start(2 * p + 1, 1)
            process(2 * p, 0)

            @pl.when(p + 1 < nch // 2)
            def _():
                start(2 * p + 2, 0)
            process(2 * p + 1, 1)
            return carry

        lax.fori_loop(0, nch // 2, body, 0)
        plsc.subcore_barrier()

        def wout(i, c):
            r0 = sid * rpt + i * CB

            @pl.when(cid == 0)
            def _():
                pltpu.sync_copy(acc.at[pl.ds(r0, CB)],
                                out0_hbm.at[pl.ds(r0, CB)])

            @pl.when(cid == 1)
            def _():
                pltpu.sync_copy(acc.at[pl.ds(r0, CB)],
                                out1_hbm.at[pl.ds(r0, CB)])
            return c
        lax.fori_loop(0, rpt // CB, wout, 0)

    return k(alpha, srcp.reshape(EPAD // CB, CB), dstp.reshape(EPAD // CB, CB),
             Vb, invd)


def kernel(h, edge_index, edge_attr, ln1_w, ln1_b, Wq, bq, Wk, bk, Wv, bv, We,
           Wskip, bskip, ga_W1, ga_b1, ga_W2, ga_b2, ga_W3, ga_b3, ln2_w, ln2_b,
           ff_W1, ff_b1, ff_W2, ff_b2, gf_W1, gf_b1, gf_W2, gf_b2, gf_W3, gf_b3):
    pad_e = EPAD - E_
    srcp = jnp.concatenate([edge_index[0], jnp.zeros((pad_e,), jnp.int32)])
    dstp = jnp.concatenate([edge_index[1], jnp.zeros((pad_e,), jnp.int32)])
    eap = jnp.concatenate(
        [edge_attr, jnp.zeros((pad_e, DE_), jnp.float32)], axis=0)

    acat, pcat, wecat, avec, bvec, gvec, cconst = _wt_call(
        Wq, Wk, We, bq.reshape(1, -1), bk.reshape(1, -1))
    hn, hnb, Vb = _hnv_call(h, ln1_w.reshape(1, -1), ln1_b.reshape(1, -1),
                            Wv, bv.reshape(1, -1))
    hns, hnd = _sc_gather(hnb, srcp, dstp)
    alpha = _alpha_call(hnd, hns, eap, acat, pcat, avec, bvec, gvec, cconst)
    acc0, acc1 = _sc_p1(alpha, eap, dstp)
    invd, ec = _norm_call(acc0, acc1, wecat)
    o0, o1 = _sc_p2(alpha, srcp, dstp, Vb, invd)
    ga = (ga_W1, ga_b1.reshape(1, -1), ga_W2, ga_b2.reshape(1, -1),
          ga_W3.reshape(1, -1), ga_b3.reshape(1, -1))
    ff = (ff_W1, ff_b1.reshape(1, -1), ff_W2, ff_b2.reshape(1, -1))
    gf = (gf_W1, gf_b1.reshape(1, -1), gf_W2, gf_b2.reshape(1, -1),
          gf_W3.reshape(1, -1), gf_b3.reshape(1, -1))
    return _final_call(hn, o0, o1, ec,
                       Wskip, bskip.reshape(1, -1), ga,
                       ln2_w.reshape(1, -1), ln2_b.reshape(1, -1), ff, gf)


# speedup vs baseline: 4.1163x; 1.0037x over previous
import functools

import jax
import jax.numpy as jnp
import numpy as np
from jax import lax
from jax.experimental import pallas as pl
from jax.experimental.pallas import tpu as pltpu
from jax.experimental.pallas import tpu_sc as plsc

N_ = 10000
E_ = 160000
D_ = 128
H_ = 8
DE_ = 16
NPAD = 10240
EPAD = 163840
NC = 2
NS = 16
NW = NC * NS
EPW = EPAD // NW
ACCW = 144
RSQD = float(1.0 / np.sqrt(D_))
NEG = -1e9

_MESH = functools.partial(
    plsc.VectorSubcoreMesh, core_axis_name="c", subcore_axis_name="s")


def _wt_body(wq, wk, we, bq, bk,
             acat, pcat, wecat, avec, bvec, gvec, cconst):
    dn = (((1,), (1,)), ((), ()))
    for h in range(H_):
        wq_h = wq[:, h * D_:(h + 1) * D_]
        wk_h = wk[:, h * D_:(h + 1) * D_]
        we_h = we[:, h * D_:(h + 1) * D_]
        bq_h = bq[:, h * D_:(h + 1) * D_]
        bk_h = bk[:, h * D_:(h + 1) * D_]
        acat[:, h * D_:(h + 1) * D_] = lax.dot_general(
            wq_h, wk_h, dn, preferred_element_type=jnp.float32).astype(
                jnp.bfloat16)
        pcat[:, h * DE_:(h + 1) * DE_] = lax.dot_general(
            wq_h, we_h, dn, preferred_element_type=jnp.float32).astype(
                jnp.bfloat16)
        wecat[h * DE_:(h + 1) * DE_, :] = we_h
        avec[:, h:h + 1] = lax.dot_general(
            wq_h, bk_h, dn, preferred_element_type=jnp.float32)
        bvec[:, h:h + 1] = lax.dot_general(
            wk_h, bq_h, dn, preferred_element_type=jnp.float32)
        gvec[:, h:h + 1] = lax.dot_general(
            we_h, bq_h, dn, preferred_element_type=jnp.float32)
        cconst[:, h:h + 1] = jnp.sum(bq_h * bk_h, axis=1, keepdims=True)


def _wt_call(Wq, Wk, We, bq2, bk2):
    full = lambda shape: pl.BlockSpec(shape, lambda: (0, 0))
    return pl.pallas_call(
        _wt_body,
        grid=(),
        in_specs=[full((D_, H_ * D_)), full((D_, H_ * D_)), full((DE_, H_ * D_)),
                  full((1, H_ * D_)), full((1, H_ * D_))],
        out_specs=[full((D_, H_ * D_)), full((D_, H_ * DE_)), full((H_ * DE_, D_)),
                   full((D_, H_)), full((D_, H_)), full((DE_, H_)), full((1, H_))],
        out_shape=[jax.ShapeDtypeStruct((D_, H_ * D_), jnp.bfloat16),
                   jax.ShapeDtypeStruct((D_, H_ * DE_), jnp.bfloat16),
                   jax.ShapeDtypeStruct((H_ * DE_, D_), jnp.float32),
                   jax.ShapeDtypeStruct((D_, H_), jnp.float32),
                   jax.ShapeDtypeStruct((D_, H_), jnp.float32),
                   jax.ShapeDtypeStruct((DE_, H_), jnp.float32),
                   jax.ShapeDtypeStruct((1, H_), jnp.float32)],
    )(Wq, Wk, We, bq2, bk2)


def _hnv_body(h_ref, lnw, lnb, wv, bv, hn_ref, hnb_ref, v_ref):
    x = h_ref[...]
    mu = jnp.mean(x, axis=1, keepdims=True)
    var = jnp.mean((x - mu) ** 2, axis=1, keepdims=True)
    hn = (x - mu) / jnp.sqrt(var + 1e-5) * lnw[...] + lnb[...]
    hn_ref[...] = hn
    hnb_ref[...] = hn.astype(jnp.bfloat16)
    v = jnp.dot(hn, wv[...], preferred_element_type=jnp.float32) + bv[...]
    v_ref[...] = v.astype(jnp.bfloat16)


def _hnv_call(h, lnw2, lnb2, Wv, bv2):
    BN = 400
    grid = (N_ // BN,)
    row = lambda shape: pl.BlockSpec(shape, lambda i: (i, 0))
    full = lambda shape: pl.BlockSpec(shape, lambda i: (0, 0))
    return pl.pallas_call(
        _hnv_body,
        grid=grid,
        in_specs=[row((BN, D_)), full((1, D_)), full((1, D_)),
                  full((D_, H_ * D_)), full((1, H_ * D_))],
        out_specs=[row((BN, D_)), row((BN, D_)), row((BN, H_ * D_))],
        out_shape=[jax.ShapeDtypeStruct((N_, D_), jnp.float32),
                   jax.ShapeDtypeStruct((N_, D_), jnp.bfloat16),
                   jax.ShapeDtypeStruct((N_, H_ * D_), jnp.bfloat16)],
    )(h, lnw2, lnb2, Wv, bv2)


_BE = 512


def _alpha_body(hnd_ref, hns_ref, ea_ref, acat, pcat, avec, bvec, gvec, cconst,
                out_ref):
    hnd = hnd_ref[...]
    hns = hns_ref[...]
    hndf = hnd.astype(jnp.float32)
    hnsf = hns.astype(jnp.float32)
    ea = ea_ref[...]
    t1 = jnp.dot(hnd, acat[...], preferred_element_type=jnp.float32)
    hns_rep = jnp.concatenate([hnsf] * H_, axis=1)
    r1 = lax.broadcasted_iota(jnp.int32, (H_ * D_, H_), 0)
    c1 = lax.broadcasted_iota(jnp.int32, (H_ * D_, H_), 1)
    s1 = ((r1 // D_) == c1).astype(jnp.float32)
    al = jnp.dot(t1 * hns_rep, s1, preferred_element_type=jnp.float32)
    t2 = jnp.dot(hnd, pcat[...], preferred_element_type=jnp.float32)
    ea_rep = jnp.concatenate([ea] * H_, axis=1)
    r2 = lax.broadcasted_iota(jnp.int32, (H_ * DE_, H_), 0)
    c2 = lax.broadcasted_iota(jnp.int32, (H_ * DE_, H_), 1)
    s2 = ((r2 // DE_) == c2).astype(jnp.float32)
    al = al + jnp.dot(t2 * ea_rep, s2, preferred_element_type=jnp.float32)
    al = (al
          + jnp.dot(hndf, avec[...], preferred_element_type=jnp.float32)
          + jnp.dot(hnsf, bvec[...], preferred_element_type=jnp.float32)
          + jnp.dot(ea, gvec[...], preferred_element_type=jnp.float32)
          + cconst[...])
    al = al * RSQD
    al = jnp.concatenate([al, jnp.full((_BE, H_), NEG, jnp.float32)], axis=1)
    i = pl.program_id(0)
    rowid = i * _BE + lax.broadcasted_iota(jnp.int32, (_BE, 1), 0)
    out_ref[...] = jnp.where(rowid < E_, al, NEG)


def _alpha_call(hnd, hns, eap, acat, pcat, avec, bvec, gvec, cconst):
    grid = (EPAD // _BE,)
    row = lambda shape: pl.BlockSpec(shape, lambda i: (i, 0))
    full = lambda shape: pl.BlockSpec(shape, lambda i: (0, 0))
    return pl.pallas_call(
        _alpha_body,
        grid=grid,
        in_specs=[row((_BE, D_)), row((_BE, D_)), row((_BE, DE_)),
                  full((D_, H_ * D_)), full((D_, H_ * DE_)),
                  full((D_, H_)), full((D_, H_)), full((DE_, H_)),
                  full((1, H_))],
        out_specs=row((_BE, 2 * H_)),
        out_shape=jax.ShapeDtypeStruct((EPAD, 2 * H_), jnp.float32),
    )(hnd, hns, eap, acat, pcat, avec, bvec, gvec, cconst)


def _norm_body(acc0, acc1, wecat, invd_ref, ec_ref):
    den = acc0[:, D_:D_ + H_] + acc1[:, D_:D_ + H_]
    inv = 1.0 / (den + 1e-16)
    t = acc0[:, 0:D_] + acc1[:, 0:D_]
    parts = [t[:, h * DE_:(h + 1) * DE_] * inv[:, h:h + 1] for h in range(H_)]
    ts = jnp.concatenate(parts, axis=1)
    ec_ref[...] = jnp.dot(ts, wecat[...], preferred_element_type=jnp.float32)
    invd_ref[...] = jnp.concatenate([inv, jnp.zeros_like(inv)], axis=1)


def _norm_call(acc0, acc1, wecat):
    BN = 512
    grid = (NPAD // BN,)
    row = lambda shape: pl.BlockSpec(shape, lambda i: (i, 0))
    full = lambda shape: pl.BlockSpec(shape, lambda i: (0, 0))
    return pl.pallas_call(
        _norm_body,
        grid=grid,
        in_specs=[row((BN, ACCW)), row((BN, ACCW)), full((H_ * DE_, D_))],
        out_specs=[row((BN, 2 * H_)), row((BN, D_))],
        out_shape=[jax.ShapeDtypeStruct((NPAD, 2 * H_), jnp.float32),
                   jax.ShapeDtypeStruct((NPAD, D_), jnp.float32)],
    )(acc0, acc1, wecat)


def _final_body(hn_ref, o0_ref, o1_ref, ec_ref, wskip, bskip,
                gaW1, gab1, gaW2, gab2, gaW3r, gab3,
                ln2w, ln2b, ffW1, ffb1, ffW2, ffb2,
                gfW1, gfb1, gfW2, gfb2, gfW3r, gfb3, out_ref):
    hn = hn_ref[...]
    p = lax.broadcasted_iota(jnp.int32, (D_, D_), 0)
    c = lax.broadcasted_iota(jnp.int32, (D_, D_), 1)
    tgt = ((p >> 5) << 5) + 2 * (p & 15) + ((p >> 4) & 1)
    perm = (c == tgt).astype(jnp.float32)
    op = jnp.dot(o0_ref[...] + o1_ref[...], perm,
                 preferred_element_type=jnp.float32)
    out = ((op + ec_ref[...]) * (1.0 / H_)
           + jnp.dot(hn, wskip[...], preferred_element_type=jnp.float32)
           + bskip[...])

    def gate(u, v, W1, b1, W2, b2, W3r, b3):
        z = jnp.concatenate([u, v, u - v], axis=1)
        a = jnp.dot(z, W1[...], preferred_element_type=jnp.float32) + b1[...]
        a = a * jax.nn.sigmoid(a)
        a = jnp.dot(a, W2[...], preferred_element_type=jnp.float32) + b2[...]
        a = a * jax.nn.sigmoid(a)
        g = jnp.sum(a * W3r[...], axis=1, keepdims=True) + b3[...]
        g = jax.nn.sigmoid(g)
        return g * u + (1 - g) * v

    h1 = gate(hn, out, gaW1, gab1, gaW2, gab2, gaW3r, gab3)
    mu = jnp.mean(h1, axis=1, keepdims=True)
    var = jnp.mean((h1 - mu) ** 2, axis=1, keepdims=True)
    h2 = (h1 - mu) / jnp.sqrt(var + 1e-5) * ln2w[...] + ln2b[...]
    ff = jnp.dot(h2, ffW1[...], preferred_element_type=jnp.float32) + ffb1[...]
    ff = ff * jax.nn.sigmoid(ff)
    ff = jnp.dot(ff, ffW2[...], preferred_element_type=jnp.float32) + ffb2[...]
    out_ref[...] = gate(h2, ff, gfW1, gfb1, gfW2, gfb2, gfW3r, gfb3)


def _final_call(hn, o0, o1, ec, Wskip, bskip2, ga, ln2w2, ln2b2, ff, gf):
    BN = 400
    grid = (N_ // BN,)
    row = lambda shape: pl.BlockSpec(shape, lambda i: (i, 0))
    full = lambda shape: pl.BlockSpec(shape, lambda i: (0, 0))
    D3, D32, D34 = 3 * D_, 3 * D_ // 2, 3 * D_ // 4
    in_specs = [row((BN, D_)), row((BN, D_)), row((BN, D_)), row((BN, D_)),
                full((D_, D_)), full((1, D_)),
                full((D3, D32)), full((1, D32)), full((D32, D34)), full((1, D34)),
                full((1, D34)), full((1, 1)),
                full((1, D_)), full((1, D_)),
                full((D_, D_)), full((1, D_)), full((D_, D_)), full((1, D_)),
                full((D3, D32)), full((1, D32)), full((D32, D34)), full((1, D34)),
                full((1, D34)), full((1, 1))]
    return pl.pallas_call(
        _final_body,
        grid=grid,
        in_specs=in_specs,
        out_specs=row((BN, D_)),
        out_shape=jax.ShapeDtypeStruct((N_, D_), jnp.float32),
    )(hn, o0, o1, ec, Wskip, bskip2, *ga, ln2w2, ln2b2, *ff, *gf)


def _sc_gather(hnb, srcp, dstp):
    CB = 128
    nch = EPW // CB

    @functools.partial(
        pl.kernel,
        out_type=(jax.ShapeDtypeStruct((EPAD, D_), jnp.bfloat16),
                  jax.ShapeDtypeStruct((EPAD, D_), jnp.bfloat16)),
        mesh=_MESH(),
        compiler_params=pltpu.CompilerParams(use_tc_tiling_on_sc=False, needs_layout_passes=False),
        scratch_types=[pltpu.VMEM((EPW // 128, 128), jnp.int32),
                       pltpu.VMEM((EPW // 128, 128), jnp.int32),
                       pltpu.VMEM((2, CB, D_), jnp.bfloat16),
                       pltpu.VMEM((2, CB, D_), jnp.bfloat16),
                       pltpu.SemaphoreType.DMA, pltpu.SemaphoreType.DMA,
                       pltpu.SemaphoreType.DMA, pltpu.SemaphoreType.DMA,
                       pltpu.SemaphoreType.DMA, pltpu.SemaphoreType.DMA,
                       pltpu.SemaphoreType.DMA, pltpu.SemaphoreType.DMA],
    )
    def k(hn_hbm, src_hbm, dst_hbm, hns_hbm, hnd_hbm,
          sidx, didx, srows, drows,
          s_s0, s_s1, s_d0, s_d1, w_s0, w_s1, w_d0, w_d1):
        wid = lax.axis_index("s") * NC + lax.axis_index("c")
        ssems = (s_s0, s_s1)
        dsems = (s_d0, s_d1)
        wssems = (w_s0, w_s1)
        wdsems = (w_d0, w_d1)

        pltpu.sync_copy(src_hbm.at[pl.ds(wid * nch, nch)], sidx)
        pltpu.sync_copy(dst_hbm.at[pl.ds(wid * nch, nch)], didx)

        def start(j, b):
            pltpu.async_copy(hn_hbm.at[sidx.at[j]], srows.at[b], ssems[b])
            pltpu.async_copy(hn_hbm.at[didx.at[j]], drows.at[b], dsems[b])

        def waitwb(b):
            base = wid * EPW
            pltpu.make_async_copy(srows.at[b], hns_hbm.at[pl.ds(base, CB)],
                                  wssems[b]).wait()
            pltpu.make_async_copy(drows.at[b], hnd_hbm.at[pl.ds(base, CB)],
                                  wdsems[b]).wait()

        def drain(j, b):
            base = wid * EPW + j * CB
            pltpu.make_async_copy(hn_hbm.at[sidx.at[j]], srows.at[b],
                                  ssems[b]).wait()
            pltpu.make_async_copy(hn_hbm.at[didx.at[j]], drows.at[b],
                                  dsems[b]).wait()
            pltpu.async_copy(srows.at[b], hns_hbm.at[pl.ds(base, CB)],
                             wssems[b])
            pltpu.async_copy(drows.at[b], hnd_hbm.at[pl.ds(base, CB)],
                             wdsems[b])

        start(0, 0)
        start(1, 1)

        def body(p, carry):
            drain(2 * p, 0)

            @pl.when(p + 1 < nch // 2)
            def _():
                waitwb(0)
                start(2 * p + 2, 0)
            drain(2 * p + 1, 1)

            @pl.when(p + 1 < nch // 2)
            def _():
                waitwb(1)
                start(2 * p + 3, 1)
            return carry

        lax.fori_loop(0, nch // 2, body, 0)
        waitwb(0)
        waitwb(1)

    return k(hnb, srcp.reshape(EPAD // CB, CB), dstp.reshape(EPAD // CB, CB))


def _sc_p1(alpha, eap, dstp):
    CB = 64
    nch = EPW // CB
    rpt = NPAD // NS

    @functools.partial(
        pl.kernel,
        out_type=(jax.ShapeDtypeStruct((NPAD, ACCW), jnp.float32),
                  jax.ShapeDtypeStruct((NPAD, ACCW), jnp.float32)),
        mesh=_MESH(),
        compiler_params=pltpu.CompilerParams(use_tc_tiling_on_sc=False, needs_layout_passes=False),
        scratch_types=[pltpu.VMEM((2, CB, 2 * H_), jnp.float32),
                       pltpu.VMEM((2, CB, DE_), jnp.float32),
                       pltpu.VMEM((nch, CB), jnp.int32),
                       pltpu.VMEM((2, CB, ACCW), jnp.float32),
                       pltpu.VMEM_SHARED((NPAD, ACCW), jnp.float32),
                       pltpu.SemaphoreType.DMA, pltpu.SemaphoreType.DMA,
                       pltpu.SemaphoreType.DMA, pltpu.SemaphoreType.DMA,
                       pltpu.SemaphoreType.DMA, pltpu.SemaphoreType.DMA],
    )
    def k(alpha_hbm, ea_hbm, dst_hbm, out0_hbm, out1_hbm,
          abuf, eabuf, didx, payload, acc, sa0, sa1, se0, se1, sc0, sc1):
        cid = lax.axis_index("c")
        sid = lax.axis_index("s")
        wid = sid * NC + cid
        asems = (sa0, sa1)
        esems = (se0, se1)
        csems = (sc0, sc1)

        pltpu.sync_copy(dst_hbm.at[pl.ds(wid * nch, nch)], didx)

        def zrow(i, c):
            for bb in range(2):
                for j in range(ACCW // 16):
                    payload[bb, i, pl.ds(j * 16, 16)] = jnp.zeros(
                        (16,), jnp.float32)
            return c
        lax.fori_loop(0, CB, zrow, 0)

        def zacc(i, c):
            pltpu.sync_copy(payload.at[0],
                            acc.at[pl.ds(sid * rpt + i * CB, CB)])
            return c
        lax.fori_loop(0, rpt // CB, zacc, 0)
        plsc.subcore_barrier()

        def start(j, b):
            base = wid * EPW + j * CB
            pltpu.async_copy(alpha_hbm.at[pl.ds(base, CB)], abuf.at[b],
                             asems[b])
            pltpu.async_copy(ea_hbm.at[pl.ds(base, CB)], eabuf.at[b],
                             esems[b])

        def waitsc(j, b):
            pltpu.make_async_copy(payload.at[b], acc.at[didx.at[j]],
                                  csems[b]).wait()

        def process(j, b):
            base = wid * EPW + j * CB
            pltpu.make_async_copy(alpha_hbm.at[pl.ds(base, CB)], abuf.at[b],
                                  asems[b]).wait()
            pltpu.make_async_copy(ea_hbm.at[pl.ds(base, CB)], eabuf.at[b],
                                  esems[b]).wait()

            def edge(e, c2):
                ex16 = jnp.exp(abuf[b, e, pl.ds(0, 16)])
                payload[b, e, pl.ds(D_, 16)] = ex16
                earow = eabuf[b, e, pl.ds(0, DE_)]
                for h in range(H_):
                    payload[b, e, pl.ds(h * DE_, DE_)] = (
                        jnp.full((16,), ex16[h]) * earow)
                return c2

            lax.fori_loop(0, CB, edge, 0)
            pltpu.async_copy(payload.at[b], acc.at[didx.at[j]], csems[b],
                             add=True)

        start(0, 0)
        start(1, 1)

        def body(p, carry):
            process(2 * p, 0)

            @pl.when(p + 1 < nch // 2)
            def _():
                start(2 * p + 2, 0)
            process(2 * p + 1, 1)

            @pl.when(p + 1 < nch // 2)
            def _():
                start(2 * p + 3, 1)

            @pl.when(p + 1 < nch // 2)
            def _():
                waitsc(2 * p, 0)
                waitsc(2 * p + 1, 1)
            return carry

        lax.fori_loop(0, nch // 2, body, 0)
        waitsc(nch - 2, 0)
        waitsc(nch - 1, 1)
        plsc.subcore_barrier()

        def wout(i, c):
            r0 = sid * rpt + i * CB

            @pl.when(cid == 0)
            def _():
                pltpu.sync_copy(acc.at[pl.ds(r0, CB)],
                                out0_hbm.at[pl.ds(r0, CB)])

            @pl.when(cid == 1)
            def _():
                pltpu.sync_copy(acc.at[pl.ds(r0, CB)],
                                out1_hbm.at[pl.ds(r0, CB)])
            return c
        lax.fori_loop(0, rpt // CB, wout, 0)

    return k(alpha, eap, dstp.reshape(EPAD // CB, CB))


def _sc_p2(alpha, srcp, dstp, Vb, invd):
    CB = 32
    nch = EPW // CB
    rpt = NPAD // NS

    @functools.partial(
        pl.kernel,
        out_type=(jax.ShapeDtypeStruct((NPAD, D_), jnp.float32),
                  jax.ShapeDtypeStruct((NPAD, D_), jnp.float32)),
        mesh=_MESH(),
        compiler_params=pltpu.CompilerParams(use_tc_tiling_on_sc=False, needs_layout_passes=False),
        scratch_types=[pltpu.VMEM((2, CB, 2 * H_), jnp.float32),
                       pltpu.VMEM((2, CB, 2 * H_), jnp.float32),
                       pltpu.VMEM((nch, CB), jnp.int32),
                       pltpu.VMEM((nch, CB), jnp.int32),
                       pltpu.VMEM((2, CB, H_ * D_), jnp.bfloat16),
                       pltpu.VMEM((CB, D_), jnp.float32),
                       pltpu.VMEM_SHARED((NPAD, D_), jnp.float32),
                       pltpu.SemaphoreType.DMA, pltpu.SemaphoreType.DMA,
                       pltpu.SemaphoreType.DMA, pltpu.SemaphoreType.DMA],
    )
    def k(alpha_hbm, src_hbm, dst_hbm, v_hbm, invd_hbm, out0_hbm, out1_hbm,
          abuf, ivbuf, sidx, didx, vrows, wpay, acc, sv0, sv1, si0, si1):
        cid = lax.axis_index("c")
        sid = lax.axis_index("s")
        wid = sid * NC + cid
        vsems = (sv0, sv1)
        isems = (si0, si1)

        pltpu.sync_copy(src_hbm.at[pl.ds(wid * nch, nch)], sidx)
        pltpu.sync_copy(dst_hbm.at[pl.ds(wid * nch, nch)], didx)

        def zrow(i, c):
            for j in range(D_ // 16):
                wpay[i, pl.ds(j * 16, 16)] = jnp.zeros((16,), jnp.float32)
            return c
        lax.fori_loop(0, CB, zrow, 0)

        def zacc(i, c):
            pltpu.sync_copy(wpay, acc.at[pl.ds(sid * rpt + i * CB, CB)])
            return c
        lax.fori_loop(0, rpt // CB, zacc, 0)
        plsc.subcore_barrier()

        def start(j, b):
            base = wid * EPW + j * CB
            pltpu.sync_copy(alpha_hbm.at[pl.ds(base, CB)], abuf.at[b])
            pltpu.async_copy(v_hbm.at[sidx.at[j]], vrows.at[b], vsems[b])
            pltpu.async_copy(invd_hbm.at[didx.at[j]], ivbuf.at[b], isems[b])

        def process(j, b):
            pltpu.make_async_copy(
                v_hbm.at[sidx.at[j]], vrows.at[b], vsems[b]).wait()
            pltpu.make_async_copy(
                invd_hbm.at[didx.at[j]], ivbuf.at[b], isems[b]).wait()

            def edge(e, c2):
                attn16 = (jnp.exp(abuf[b, e, pl.ds(0, 16)])
                          * ivbuf[b, e, pl.ds(0, 16)])
                accs = [jnp.zeros((16,), jnp.float32)
                        for _ in range(D_ // 16)]
                for h in range(H_):
                    avf = jnp.full((16,), attn16[h])
                    avv = plsc.pack(avf, avf,
                                    format=plsc.PackFormat.INTERLEAVED)
                    for g in range(D_ // 32):
                        x32 = vrows[b, e, pl.ds(h * D_ + g * 32, 32)]
                        lo, hi = plsc.unpack(
                            x32 * avv, format=plsc.PackFormat.INTERLEAVED)
                        accs[2 * g] = accs[2 * g] + lo
                        accs[2 * g + 1] = accs[2 * g + 1] + hi
                for dv in range(D_ // 16):
                    wpay[e, pl.ds(dv * 16, 16)] = accs[dv]
                return c2

            lax.fori_loop(0, CB, edge, 0)
            pltpu.sync_copy(wpay, acc.at[didx.at[j]], add=True)

        start(0, 0)

        def body(p, carry):
            start(2 * p + 1, 1)
            process(2 * p, 0)

            @pl.when(p + 1 < nch // 2)
            def _():
                start(2 * p + 2, 0)
            process(2 * p + 1, 1)
            return carry

        lax.fori_loop(0, nch // 2, body, 0)
        plsc.subcore_barrier()

        def wout(i, c):
            r0 = sid * rpt + i * CB

            @pl.when(cid == 0)
            def _():
                pltpu.sync_copy(acc.at[pl.ds(r0, CB)],
                                out0_hbm.at[pl.ds(r0, CB)])

            @pl.when(cid == 1)
            def _():
                pltpu.sync_copy(acc.at[pl.ds(r0, CB)],
                                out1_hbm.at[pl.ds(r0, CB)])
            return c
        lax.fori_loop(0, rpt // CB, wout, 0)

    return k(alpha, srcp.reshape(EPAD // CB, CB), dstp.reshape(EPAD // CB, CB),
             Vb, invd)


def kernel(h, edge_index, edge_attr, ln1_w, ln1_b, Wq, bq, Wk, bk, Wv, bv, We,
           Wskip, bskip, ga_W1, ga_b1, ga_W2, ga_b2, ga_W3, ga_b3, ln2_w, ln2_b,
           ff_W1, ff_b1, ff_W2, ff_b2, gf_W1, gf_b1, gf_W2, gf_b2, gf_W3, gf_b3):
    pad_e = EPAD - E_
    srcp = jnp.concatenate([edge_index[0], jnp.zeros((pad_e,), jnp.int32)])
    dstp = jnp.concatenate([edge_index[1], jnp.zeros((pad_e,), jnp.int32)])
    eap = jnp.concatenate(
        [edge_attr, jnp.zeros((pad_e, DE_), jnp.float32)], axis=0)

    acat, pcat, wecat, avec, bvec, gvec, cconst = _wt_call(
        Wq, Wk, We, bq.reshape(1, -1), bk.reshape(1, -1))
    hn, hnb, Vb = _hnv_call(h, ln1_w.reshape(1, -1), ln1_b.reshape(1, -1),
                            Wv, bv.reshape(1, -1))
    hns, hnd = _sc_gather(hnb, srcp, dstp)
    alpha = _alpha_call(hnd, hns, eap, acat, pcat, avec, bvec, gvec, cconst)
    acc0, acc1 = _sc_p1(alpha, eap, dstp)
    invd, ec = _norm_call(acc0, acc1, wecat)
    o0, o1 = _sc_p2(alpha, srcp, dstp, Vb, invd)
    ga = (ga_W1, ga_b1.reshape(1, -1), ga_W2, ga_b2.reshape(1, -1),
          ga_W3.reshape(1, -1), ga_b3.reshape(1, -1))
    ff = (ff_W1, ff_b1.reshape(1, -1), ff_W2, ff_b2.reshape(1, -1))
    gf = (gf_W1, gf_b1.reshape(1, -1), gf_W2, gf_b2.reshape(1, -1),
          gf_W3.reshape(1, -1), gf_b3.reshape(1, -1))
    return _final_call(hn, o0, o1, ec,
                       Wskip, bskip.reshape(1, -1), ga,
                       ln2_w.reshape(1, -1), ln2_b.reshape(1, -1), ff, gf)


# speedup vs baseline: 4.2846x; 1.0409x over previous
import functools

import jax
import jax.numpy as jnp
import numpy as np
from jax import lax
from jax.experimental import pallas as pl
from jax.experimental.pallas import tpu as pltpu
from jax.experimental.pallas import tpu_sc as plsc

N_ = 10000
E_ = 160000
D_ = 128
H_ = 8
DE_ = 16
NPAD = 10240
EPAD = 163840
NC = 2
NS = 16
NW = NC * NS
EPW = EPAD // NW
ACCW = 144
RSQD = float(1.0 / np.sqrt(D_))
NEG = -1e9

_MESH = functools.partial(
    plsc.VectorSubcoreMesh, core_axis_name="c", subcore_axis_name="s")


def _wt_body(wq, wk, we, bq, bk,
             acat, pcat, wecat, avec, bvec, gvec, cconst):
    dn = (((1,), (1,)), ((), ()))
    for h in range(H_):
        wq_h = wq[:, h * D_:(h + 1) * D_]
        wk_h = wk[:, h * D_:(h + 1) * D_]
        we_h = we[:, h * D_:(h + 1) * D_]
        bq_h = bq[:, h * D_:(h + 1) * D_]
        bk_h = bk[:, h * D_:(h + 1) * D_]
        acat[:, h * D_:(h + 1) * D_] = lax.dot_general(
            wq_h, wk_h, dn, preferred_element_type=jnp.float32).astype(
                jnp.bfloat16)
        pcat[:, h * DE_:(h + 1) * DE_] = lax.dot_general(
            wq_h, we_h, dn, preferred_element_type=jnp.float32).astype(
                jnp.bfloat16)
        wecat[h * DE_:(h + 1) * DE_, :] = we_h
        avec[:, h:h + 1] = lax.dot_general(
            wq_h, bk_h, dn, preferred_element_type=jnp.float32)
        bvec[:, h:h + 1] = lax.dot_general(
            wk_h, bq_h, dn, preferred_element_type=jnp.float32)
        gvec[:, h:h + 1] = lax.dot_general(
            we_h, bq_h, dn, preferred_element_type=jnp.float32)
        cconst[:, h:h + 1] = jnp.sum(bq_h * bk_h, axis=1, keepdims=True)


def _wt_call(Wq, Wk, We, bq2, bk2):
    full = lambda shape: pl.BlockSpec(shape, lambda: (0, 0))
    return pl.pallas_call(
        _wt_body,
        grid=(),
        in_specs=[full((D_, H_ * D_)), full((D_, H_ * D_)), full((DE_, H_ * D_)),
                  full((1, H_ * D_)), full((1, H_ * D_))],
        out_specs=[full((D_, H_ * D_)), full((D_, H_ * DE_)), full((H_ * DE_, D_)),
                   full((D_, H_)), full((D_, H_)), full((DE_, H_)), full((1, H_))],
        out_shape=[jax.ShapeDtypeStruct((D_, H_ * D_), jnp.bfloat16),
                   jax.ShapeDtypeStruct((D_, H_ * DE_), jnp.bfloat16),
                   jax.ShapeDtypeStruct((H_ * DE_, D_), jnp.float32),
                   jax.ShapeDtypeStruct((D_, H_), jnp.float32),
                   jax.ShapeDtypeStruct((D_, H_), jnp.float32),
                   jax.ShapeDtypeStruct((DE_, H_), jnp.float32),
                   jax.ShapeDtypeStruct((1, H_), jnp.float32)],
    )(Wq, Wk, We, bq2, bk2)


def _hnv_body(h_ref, lnw, lnb, wv, bv, hn_ref, hnb_ref, v_ref):
    x = h_ref[...]
    mu = jnp.mean(x, axis=1, keepdims=True)
    var = jnp.mean((x - mu) ** 2, axis=1, keepdims=True)
    hn = (x - mu) / jnp.sqrt(var + 1e-5) * lnw[...] + lnb[...]
    hn_ref[...] = hn
    hnb_ref[...] = hn.astype(jnp.bfloat16)
    v = jnp.dot(hn, wv[...], preferred_element_type=jnp.float32) + bv[...]
    v_ref[...] = v.astype(jnp.bfloat16)


def _hnv_call(h, lnw2, lnb2, Wv, bv2):
    BN = 400
    grid = (N_ // BN,)
    row = lambda shape: pl.BlockSpec(shape, lambda i: (i, 0))
    full = lambda shape: pl.BlockSpec(shape, lambda i: (0, 0))
    return pl.pallas_call(
        _hnv_body,
        grid=grid,
        in_specs=[row((BN, D_)), full((1, D_)), full((1, D_)),
                  full((D_, H_ * D_)), full((1, H_ * D_))],
        out_specs=[row((BN, D_)), row((BN, D_)), row((BN, H_ * D_))],
        out_shape=[jax.ShapeDtypeStruct((N_, D_), jnp.float32),
                   jax.ShapeDtypeStruct((N_, D_), jnp.bfloat16),
                   jax.ShapeDtypeStruct((N_, H_ * D_), jnp.bfloat16)],
    )(h, lnw2, lnb2, Wv, bv2)


_BE = 1024


def _alpha_body(hnd_ref, hns_ref, ea_ref, acat, pcat, avec, bvec, gvec, cconst,
                out_ref):
    hnd = hnd_ref[...]
    hns = hns_ref[...]
    hndf = hnd.astype(jnp.float32)
    hnsf = hns.astype(jnp.float32)
    ea = ea_ref[...]
    t1 = jnp.dot(hnd, acat[...], preferred_element_type=jnp.float32)
    hns_rep = jnp.concatenate([hnsf] * H_, axis=1)
    r1 = lax.broadcasted_iota(jnp.int32, (H_ * D_, H_), 0)
    c1 = lax.broadcasted_iota(jnp.int32, (H_ * D_, H_), 1)
    s1 = ((r1 // D_) == c1).astype(jnp.float32)
    al = jnp.dot(t1 * hns_rep, s1, preferred_element_type=jnp.float32)
    t2 = jnp.dot(hnd, pcat[...], preferred_element_type=jnp.float32)
    ea_rep = jnp.concatenate([ea] * H_, axis=1)
    r2 = lax.broadcasted_iota(jnp.int32, (H_ * DE_, H_), 0)
    c2 = lax.broadcasted_iota(jnp.int32, (H_ * DE_, H_), 1)
    s2 = ((r2 // DE_) == c2).astype(jnp.float32)
    al = al + jnp.dot(t2 * ea_rep, s2, preferred_element_type=jnp.float32)
    al = (al
          + jnp.dot(hndf, avec[...], preferred_element_type=jnp.float32)
          + jnp.dot(hnsf, bvec[...], preferred_element_type=jnp.float32)
          + jnp.dot(ea, gvec[...], preferred_element_type=jnp.float32)
          + cconst[...])
    al = al * RSQD
    al = jnp.concatenate([al, jnp.full((_BE, H_), NEG, jnp.float32)], axis=1)
    i = pl.program_id(0)
    rowid = i * _BE + lax.broadcasted_iota(jnp.int32, (_BE, 1), 0)
    out_ref[...] = jnp.where(rowid < E_, al, NEG)


def _alpha_call(hnd, hns, eap, acat, pcat, avec, bvec, gvec, cconst):
    grid = (EPAD // _BE,)
    row = lambda shape: pl.BlockSpec(shape, lambda i: (i, 0))
    full = lambda shape: pl.BlockSpec(shape, lambda i: (0, 0))
    return pl.pallas_call(
        _alpha_body,
        grid=grid,
        in_specs=[row((_BE, D_)), row((_BE, D_)), row((_BE, DE_)),
                  full((D_, H_ * D_)), full((D_, H_ * DE_)),
                  full((D_, H_)), full((D_, H_)), full((DE_, H_)),
                  full((1, H_))],
        out_specs=row((_BE, 2 * H_)),
        out_shape=jax.ShapeDtypeStruct((EPAD, 2 * H_), jnp.float32),
    )(hnd, hns, eap, acat, pcat, avec, bvec, gvec, cconst)


def _norm_body(acc0, acc1, wecat, invd_ref, ec_ref):
    den = acc0[:, D_:D_ + H_] + acc1[:, D_:D_ + H_]
    inv = 1.0 / (den + 1e-16)
    t = acc0[:, 0:D_] + acc1[:, 0:D_]
    parts = [t[:, h * DE_:(h + 1) * DE_] * inv[:, h:h + 1] for h in range(H_)]
    ts = jnp.concatenate(parts, axis=1)
    ec_ref[...] = jnp.dot(ts, wecat[...], preferred_element_type=jnp.float32)
    invd_ref[...] = jnp.concatenate([inv, jnp.zeros_like(inv)], axis=1)


def _norm_call(acc0, acc1, wecat):
    BN = 512
    grid = (NPAD // BN,)
    row = lambda shape: pl.BlockSpec(shape, lambda i: (i, 0))
    full = lambda shape: pl.BlockSpec(shape, lambda i: (0, 0))
    return pl.pallas_call(
        _norm_body,
        grid=grid,
        in_specs=[row((BN, ACCW)), row((BN, ACCW)), full((H_ * DE_, D_))],
        out_specs=[row((BN, 2 * H_)), row((BN, D_))],
        out_shape=[jax.ShapeDtypeStruct((NPAD, 2 * H_), jnp.float32),
                   jax.ShapeDtypeStruct((NPAD, D_), jnp.float32)],
    )(acc0, acc1, wecat)


def _final_body(hn_ref, o0_ref, o1_ref, ec_ref, wskip, bskip,
                gaW1, gab1, gaW2, gab2, gaW3r, gab3,
                ln2w, ln2b, ffW1, ffb1, ffW2, ffb2,
                gfW1, gfb1, gfW2, gfb2, gfW3r, gfb3, out_ref):
    hn = hn_ref[...]
    p = lax.broadcasted_iota(jnp.int32, (D_, D_), 0)
    c = lax.broadcasted_iota(jnp.int32, (D_, D_), 1)
    tgt = ((p >> 5) << 5) + 2 * (p & 15) + ((p >> 4) & 1)
    perm = (c == tgt).astype(jnp.float32)
    op = jnp.dot(o0_ref[...] + o1_ref[...], perm,
                 preferred_element_type=jnp.float32)
    out = ((op + ec_ref[...]) * (1.0 / H_)
           + jnp.dot(hn, wskip[...], preferred_element_type=jnp.float32)
           + bskip[...])

    def gate(u, v, W1, b1, W2, b2, W3r, b3):
        z = jnp.concatenate([u, v, u - v], axis=1)
        a = jnp.dot(z, W1[...], preferred_element_type=jnp.float32) + b1[...]
        a = a * jax.nn.sigmoid(a)
        a = jnp.dot(a, W2[...], preferred_element_type=jnp.float32) + b2[...]
        a = a * jax.nn.sigmoid(a)
        g = jnp.sum(a * W3r[...], axis=1, keepdims=True) + b3[...]
        g = jax.nn.sigmoid(g)
        return g * u + (1 - g) * v

    h1 = gate(hn, out, gaW1, gab1, gaW2, gab2, gaW3r, gab3)
    mu = jnp.mean(h1, axis=1, keepdims=True)
    var = jnp.mean((h1 - mu) ** 2, axis=1, keepdims=True)
    h2 = (h1 - mu) / jnp.sqrt(var + 1e-5) * ln2w[...] + ln2b[...]
    ff = jnp.dot(h2, ffW1[...], preferred_element_type=jnp.float32) + ffb1[...]
    ff = ff * jax.nn.sigmoid(ff)
    ff = jnp.dot(ff, ffW2[...], preferred_element_type=jnp.float32) + ffb2[...]
    out_ref[...] = gate(h2, ff, gfW1, gfb1, gfW2, gfb2, gfW3r, gfb3)


def _final_call(hn, o0, o1, ec, Wskip, bskip2, ga, ln2w2, ln2b2, ff, gf):
    BN = 400
    grid = (N_ // BN,)
    row = lambda shape: pl.BlockSpec(shape, lambda i: (i, 0))
    full = lambda shape: pl.BlockSpec(shape, lambda i: (0, 0))
    D3, D32, D34 = 3 * D_, 3 * D_ // 2, 3 * D_ // 4
    in_specs = [row((BN, D_)), row((BN, D_)), row((BN, D_)), row((BN, D_)),
                full((D_, D_)), full((1, D_)),
                full((D3, D32)), full((1, D32)), full((D32, D34)), full((1, D34)),
                full((1, D34)), full((1, 1)),
                full((1, D_)), full((1, D_)),
                full((D_, D_)), full((1, D_)), full((D_, D_)), full((1, D_)),
                full((D3, D32)), full((1, D32)), full((D32, D34)), full((1, D34)),
                full((1, D34)), full((1, 1))]
    return pl.pallas_call(
        _final_body,
        grid=grid,
        in_specs=in_specs,
        out_specs=row((BN, D_)),
        out_shape=jax.ShapeDtypeStruct((N_, D_), jnp.float32),
    )(hn, o0, o1, ec, Wskip, bskip2, *ga, ln2w2, ln2b2, *ff, *gf)


def _sc_gather(hnb, srcp, dstp):
    CB = 128
    nch = EPW // CB

    @functools.partial(
        pl.kernel,
        out_type=(jax.ShapeDtypeStruct((EPAD, D_), jnp.bfloat16),
                  jax.ShapeDtypeStruct((EPAD, D_), jnp.bfloat16)),
        mesh=_MESH(),
        compiler_params=pltpu.CompilerParams(use_tc_tiling_on_sc=False, needs_layout_passes=False),
        scratch_types=[pltpu.VMEM((EPW // 128, 128), jnp.int32),
                       pltpu.VMEM((EPW // 128, 128), jnp.int32),
                       pltpu.VMEM((2, CB, D_), jnp.bfloat16),
                       pltpu.VMEM((2, CB, D_), jnp.bfloat16),
                       pltpu.SemaphoreType.DMA, pltpu.SemaphoreType.DMA,
                       pltpu.SemaphoreType.DMA, pltpu.SemaphoreType.DMA,
                       pltpu.SemaphoreType.DMA, pltpu.SemaphoreType.DMA,
                       pltpu.SemaphoreType.DMA, pltpu.SemaphoreType.DMA],
    )
    def k(hn_hbm, src_hbm, dst_hbm, hns_hbm, hnd_hbm,
          sidx, didx, srows, drows,
          s_s0, s_s1, s_d0, s_d1, w_s0, w_s1, w_d0, w_d1):
        wid = lax.axis_index("s") * NC + lax.axis_index("c")
        ssems = (s_s0, s_s1)
        dsems = (s_d0, s_d1)
        wssems = (w_s0, w_s1)
        wdsems = (w_d0, w_d1)

        pltpu.sync_copy(src_hbm.at[pl.ds(wid * nch, nch)], sidx)
        pltpu.sync_copy(dst_hbm.at[pl.ds(wid * nch, nch)], didx)

        def start(j, b):
            pltpu.async_copy(hn_hbm.at[sidx.at[j]], srows.at[b], ssems[b])
            pltpu.async_copy(hn_hbm.at[didx.at[j]], drows.at[b], dsems[b])

        def waitwb(b):
            base = wid * EPW
            pltpu.make_async_copy(srows.at[b], hns_hbm.at[pl.ds(base, CB)],
                                  wssems[b]).wait()
            pltpu.make_async_copy(drows.at[b], hnd_hbm.at[pl.ds(base, CB)],
                                  wdsems[b]).wait()

        def drain(j, b):
            base = wid * EPW + j * CB
            pltpu.make_async_copy(hn_hbm.at[sidx.at[j]], srows.at[b],
                                  ssems[b]).wait()
            pltpu.make_async_copy(hn_hbm.at[didx.at[j]], drows.at[b],
                                  dsems[b]).wait()
            pltpu.async_copy(srows.at[b], hns_hbm.at[pl.ds(base, CB)],
                             wssems[b])
            pltpu.async_copy(drows.at[b], hnd_hbm.at[pl.ds(base, CB)],
                             wdsems[b])

        start(0, 0)
        start(1, 1)

        def body(p, carry):
            drain(2 * p, 0)

            @pl.when(p + 1 < nch // 2)
            def _():
                waitwb(0)
                start(2 * p + 2, 0)
            drain(2 * p + 1, 1)

            @pl.when(p + 1 < nch // 2)
            def _():
                waitwb(1)
                start(2 * p + 3, 1)
            return carry

        lax.fori_loop(0, nch // 2, body, 0)
        waitwb(0)
        waitwb(1)

    return k(hnb, srcp.reshape(EPAD // CB, CB), dstp.reshape(EPAD // CB, CB))


def _sc_p1(alpha, eap, dstp):
    CB = 64
    nch = EPW // CB
    rpt = NPAD // NS

    @functools.partial(
        pl.kernel,
        out_type=(jax.ShapeDtypeStruct((NPAD, ACCW), jnp.float32),
                  jax.ShapeDtypeStruct((NPAD, ACCW), jnp.float32)),
        mesh=_MESH(),
        compiler_params=pltpu.CompilerParams(use_tc_tiling_on_sc=False, needs_layout_passes=False),
        scratch_types=[pltpu.VMEM((2, CB, 2 * H_), jnp.float32),
                       pltpu.VMEM((2, CB, DE_), jnp.float32),
                       pltpu.VMEM((nch, CB), jnp.int32),
                       pltpu.VMEM((2, CB, ACCW), jnp.float32),
                       pltpu.VMEM_SHARED((NPAD, ACCW), jnp.float32),
                       pltpu.SemaphoreType.DMA, pltpu.SemaphoreType.DMA,
                       pltpu.SemaphoreType.DMA, pltpu.SemaphoreType.DMA,
                       pltpu.SemaphoreType.DMA, pltpu.SemaphoreType.DMA],
    )
    def k(alpha_hbm, ea_hbm, dst_hbm, out0_hbm, out1_hbm,
          abuf, eabuf, didx, payload, acc, sa0, sa1, se0, se1, sc0, sc1):
        cid = lax.axis_index("c")
        sid = lax.axis_index("s")
        wid = sid * NC + cid
        asems = (sa0, sa1)
        esems = (se0, se1)
        csems = (sc0, sc1)

        pltpu.sync_copy(dst_hbm.at[pl.ds(wid * nch, nch)], didx)

        def zrow(i, c):
            for bb in range(2):
                for j in range(ACCW // 16):
                    payload[bb, i, pl.ds(j * 16, 16)] = jnp.zeros(
                        (16,), jnp.float32)
            return c
        lax.fori_loop(0, CB, zrow, 0)

        def zacc(i, c):
            pltpu.sync_copy(payload.at[0],
                            acc.at[pl.ds(sid * rpt + i * CB, CB)])
            return c
        lax.fori_loop(0, rpt // CB, zacc, 0)
        plsc.subcore_barrier()

        def start(j, b):
            base = wid * EPW + j * CB
            pltpu.async_copy(alpha_hbm.at[pl.ds(base, CB)], abuf.at[b],
                             asems[b])
            pltpu.async_copy(ea_hbm.at[pl.ds(base, CB)], eabuf.at[b],
                             esems[b])

        def waitsc(j, b):
            pltpu.make_async_copy(payload.at[b], acc.at[didx.at[j]],
                                  csems[b]).wait()

        def process(j, b):
            base = wid * EPW + j * CB
            pltpu.make_async_copy(alpha_hbm.at[pl.ds(base, CB)], abuf.at[b],
                                  asems[b]).wait()
            pltpu.make_async_copy(ea_hbm.at[pl.ds(base, CB)], eabuf.at[b],
                                  esems[b]).wait()

            def edge(e, c2):
                ex16 = jnp.exp(abuf[b, e, pl.ds(0, 16)])
                payload[b, e, pl.ds(D_, 16)] = ex16
                earow = eabuf[b, e, pl.ds(0, DE_)]
                for h in range(H_):
                    payload[b, e, pl.ds(h * DE_, DE_)] = (
                        jnp.full((16,), ex16[h]) * earow)
                return c2

            lax.fori_loop(0, CB, edge, 0)
            pltpu.async_copy(payload.at[b], acc.at[didx.at[j]], csems[b],
                             add=True)

        start(0, 0)
        start(1, 1)

        def body(p, carry):
            process(2 * p, 0)

            @pl.when(p + 1 < nch // 2)
            def _():
                start(2 * p + 2, 0)
            process(2 * p + 1, 1)

            @pl.when(p + 1 < nch // 2)
            def _():
                start(2 * p + 3, 1)

            @pl.when(p + 1 < nch // 2)
            def _():
                waitsc(2 * p, 0)
                waitsc(2 * p + 1, 1)
            return carry

        lax.fori_loop(0, nch // 2, body, 0)
        waitsc(nch - 2, 0)
        waitsc(nch - 1, 1)
        plsc.subcore_barrier()

        def wout(i, c):
            r0 = sid * rpt + i * CB

            @pl.when(cid == 0)
            def _():
                pltpu.sync_copy(acc.at[pl.ds(r0, CB)],
                                out0_hbm.at[pl.ds(r0, CB)])

            @pl.when(cid == 1)
            def _():
                pltpu.sync_copy(acc.at[pl.ds(r0, CB)],
                                out1_hbm.at[pl.ds(r0, CB)])
            return c
        lax.fori_loop(0, rpt // CB, wout, 0)

    return k(alpha, eap, dstp.reshape(EPAD // CB, CB))


def _sc_p2(alpha, srcp, dstp, Vb, invd):
    CB = 32
    nch = EPW // CB
    rpt = NPAD // NS

    @functools.partial(
        pl.kernel,
        out_type=(jax.ShapeDtypeStruct((NPAD, D_), jnp.float32),
                  jax.ShapeDtypeStruct((NPAD, D_), jnp.float32)),
        mesh=_MESH(),
        compiler_params=pltpu.CompilerParams(use_tc_tiling_on_sc=False, needs_layout_passes=False),
        scratch_types=[pltpu.VMEM((2, CB, 2 * H_), jnp.float32),
                       pltpu.VMEM((2, CB, 2 * H_), jnp.float32),
                       pltpu.VMEM((nch, CB), jnp.int32),
                       pltpu.VMEM((nch, CB), jnp.int32),
                       pltpu.VMEM((2, CB, H_ * D_), jnp.bfloat16),
                       pltpu.VMEM((CB, D_), jnp.float32),
                       pltpu.VMEM_SHARED((NPAD, D_), jnp.float32),
                       pltpu.SemaphoreType.DMA, pltpu.SemaphoreType.DMA,
                       pltpu.SemaphoreType.DMA, pltpu.SemaphoreType.DMA],
    )
    def k(alpha_hbm, src_hbm, dst_hbm, v_hbm, invd_hbm, out0_hbm, out1_hbm,
          abuf, ivbuf, sidx, didx, vrows, wpay, acc, sv0, sv1, si0, si1):
        cid = lax.axis_index("c")
        sid = lax.axis_index("s")
        wid = sid * NC + cid
        vsems = (sv0, sv1)
        isems = (si0, si1)

        pltpu.sync_copy(src_hbm.at[pl.ds(wid * nch, nch)], sidx)
        pltpu.sync_copy(dst_hbm.at[pl.ds(wid * nch, nch)], didx)

        def zrow(i, c):
            for j in range(D_ // 16):
                wpay[i, pl.ds(j * 16, 16)] = jnp.zeros((16,), jnp.float32)
            return c
        lax.fori_loop(0, CB, zrow, 0)

        def zacc(i, c):
            pltpu.sync_copy(wpay, acc.at[pl.ds(sid * rpt + i * CB, CB)])
            return c
        lax.fori_loop(0, rpt // CB, zacc, 0)
        plsc.subcore_barrier()

        def start(j, b):
            base = wid * EPW + j * CB
            pltpu.sync_copy(alpha_hbm.at[pl.ds(base, CB)], abuf.at[b])
            pltpu.async_copy(v_hbm.at[sidx.at[j]], vrows.at[b], vsems[b])
            pltpu.async_copy(invd_hbm.at[didx.at[j]], ivbuf.at[b], isems[b])

        def process(j, b):
            pltpu.make_async_copy(
                v_hbm.at[sidx.at[j]], vrows.at[b], vsems[b]).wait()
            pltpu.make_async_copy(
                invd_hbm.at[didx.at[j]], ivbuf.at[b], isems[b]).wait()

            def edge(e, c2):
                attn16 = (jnp.exp(abuf[b, e, pl.ds(0, 16)])
                          * ivbuf[b, e, pl.ds(0, 16)])
                accs = [jnp.zeros((16,), jnp.float32)
                        for _ in range(D_ // 16)]
                for h in range(H_):
                    avf = jnp.full((16,), attn16[h])
                    avv = plsc.pack(avf, avf,
                                    format=plsc.PackFormat.INTERLEAVED)
                    for g in range(D_ // 32):
                        x32 = vrows[b, e, pl.ds(h * D_ + g * 32, 32)]
                        lo, hi = plsc.unpack(
                            x32 * avv, format=plsc.PackFormat.INTERLEAVED)
                        accs[2 * g] = accs[2 * g] + lo
                        accs[2 * g + 1] = accs[2 * g + 1] + hi
                for dv in range(D_ // 16):
                    wpay[e, pl.ds(dv * 16, 16)] = accs[dv]
                return c2

            lax.fori_loop(0, CB, edge, 0)
            pltpu.sync_copy(wpay, acc.at[didx.at[j]], add=True)

        start(0, 0)

        def body(p, carry):
            start(2 * p + 1, 1)
            process(2 * p, 0)

            @pl.when(p + 1 < nch // 2)
            def _():
                start(2 * p + 2, 0)
            process(2 * p + 1, 1)
            return carry

        lax.fori_loop(0, nch // 2, body, 0)
        plsc.subcore_barrier()

        def wout(i, c):
            r0 = sid * rpt + i * CB

            @pl.when(cid == 0)
            def _():
                pltpu.sync_copy(acc.at[pl.ds(r0, CB)],
                                out0_hbm.at[pl.ds(r0, CB)])

            @pl.when(cid == 1)
            def _():
                pltpu.sync_copy(acc.at[pl.ds(r0, CB)],
                                out1_hbm.at[pl.ds(r0, CB)])
            return c
        lax.fori_loop(0, rpt // CB, wout, 0)

    return k(alpha, srcp.reshape(EPAD // CB, CB), dstp.reshape(EPAD // CB, CB),
             Vb, invd)


def kernel(h, edge_index, edge_attr, ln1_w, ln1_b, Wq, bq, Wk, bk, Wv, bv, We,
           Wskip, bskip, ga_W1, ga_b1, ga_W2, ga_b2, ga_W3, ga_b3, ln2_w, ln2_b,
           ff_W1, ff_b1, ff_W2, ff_b2, gf_W1, gf_b1, gf_W2, gf_b2, gf_W3, gf_b3):
    pad_e = EPAD - E_
    srcp = jnp.concatenate([edge_index[0], jnp.zeros((pad_e,), jnp.int32)])
    dstp = jnp.concatenate([edge_index[1], jnp.zeros((pad_e,), jnp.int32)])
    eap = jnp.concatenate(
        [edge_attr, jnp.zeros((pad_e, DE_), jnp.float32)], axis=0)

    acat, pcat, wecat, avec, bvec, gvec, cconst = _wt_call(
        Wq, Wk, We, bq.reshape(1, -1), bk.reshape(1, -1))
    hn, hnb, Vb = _hnv_call(h, ln1_w.reshape(1, -1), ln1_b.reshape(1, -1),
                            Wv, bv.reshape(1, -1))
    hns, hnd = _sc_gather(hnb, srcp, dstp)
    alpha = _alpha_call(hnd, hns, eap, acat, pcat, avec, bvec, gvec, cconst)
    acc0, acc1 = _sc_p1(alpha, eap, dstp)
    invd, ec = _norm_call(acc0, acc1, wecat)
    o0, o1 = _sc_p2(alpha, srcp, dstp, Vb, invd)
    ga = (ga_W1, ga_b1.reshape(1, -1), ga_W2, ga_b2.reshape(1, -1),
          ga_W3.reshape(1, -1), ga_b3.reshape(1, -1))
    ff = (ff_W1, ff_b1.reshape(1, -1), ff_W2, ff_b2.reshape(1, -1))
    gf = (gf_W1, gf_b1.reshape(1, -1), gf_W2, gf_b2.reshape(1, -1),
          gf_W3.reshape(1, -1), gf_b3.reshape(1, -1))
    return _final_call(hn, o0, o1, ec,
                       Wskip, bskip.reshape(1, -1), ga,
                       ln2_w.reshape(1, -1), ln2_b.reshape(1, -1), ff, gf)


# speedup vs baseline: 4.4299x; 1.0339x over previous
import functools

import jax
import jax.numpy as jnp
import numpy as np
from jax import lax
from jax.experimental import pallas as pl
from jax.experimental.pallas import tpu as pltpu
from jax.experimental.pallas import tpu_sc as plsc

N_ = 10000
E_ = 160000
D_ = 128
H_ = 8
DE_ = 16
NPAD = 10240
EPAD = 163840
NC = 2
NS = 16
NW = NC * NS
EPW = EPAD // NW
ACCW = 144
RSQD = float(1.0 / np.sqrt(D_))
NEG = -1e9

_MESH = functools.partial(
    plsc.VectorSubcoreMesh, core_axis_name="c", subcore_axis_name="s")


def _wt_body(wq, wk, we, bq, bk,
             acat, pcat, wecat, avec, bvec, gvec, cconst):
    dn = (((1,), (1,)), ((), ()))
    for h in range(H_):
        wq_h = wq[:, h * D_:(h + 1) * D_]
        wk_h = wk[:, h * D_:(h + 1) * D_]
        we_h = we[:, h * D_:(h + 1) * D_]
        bq_h = bq[:, h * D_:(h + 1) * D_]
        bk_h = bk[:, h * D_:(h + 1) * D_]
        acat[:, h * D_:(h + 1) * D_] = lax.dot_general(
            wq_h, wk_h, dn, preferred_element_type=jnp.float32).astype(
                jnp.bfloat16)
        pcat[:, h * DE_:(h + 1) * DE_] = lax.dot_general(
            wq_h, we_h, dn, preferred_element_type=jnp.float32).astype(
                jnp.bfloat16)
        wecat[h * DE_:(h + 1) * DE_, :] = we_h
        avec[:, h:h + 1] = lax.dot_general(
            wq_h, bk_h, dn, preferred_element_type=jnp.float32)
        bvec[:, h:h + 1] = lax.dot_general(
            wk_h, bq_h, dn, preferred_element_type=jnp.float32)
        gvec[:, h:h + 1] = lax.dot_general(
            we_h, bq_h, dn, preferred_element_type=jnp.float32)
        cconst[:, h:h + 1] = jnp.sum(bq_h * bk_h, axis=1, keepdims=True)


def _hnv_body(h_ref, lnw, lnb, wv, bv, wq, wk, we, bq, bk,
              hn_ref, hnb_ref, v_ref,
              acat, pcat, wecat, avec, bvec, gvec, cconst):
    @pl.when(pl.program_id(0) == 0)
    def _():
        _wt_body(wq, wk, we, bq, bk,
                 acat, pcat, wecat, avec, bvec, gvec, cconst)
    x = h_ref[...]
    mu = jnp.mean(x, axis=1, keepdims=True)
    var = jnp.mean((x - mu) ** 2, axis=1, keepdims=True)
    hn = (x - mu) / jnp.sqrt(var + 1e-5) * lnw[...] + lnb[...]
    hn_ref[...] = hn
    hnb_ref[...] = hn.astype(jnp.bfloat16)
    v = jnp.dot(hn, wv[...], preferred_element_type=jnp.float32) + bv[...]
    v_ref[...] = v.astype(jnp.bfloat16)


def _hnv_call(h, lnw2, lnb2, Wv, bv2, Wq, Wk, We, bq2, bk2):
    BN = 400
    grid = (N_ // BN,)
    row = lambda shape: pl.BlockSpec(shape, lambda i: (i, 0))
    full = lambda shape: pl.BlockSpec(shape, lambda i: (0, 0))
    return pl.pallas_call(
        _hnv_body,
        grid=grid,
        in_specs=[row((BN, D_)), full((1, D_)), full((1, D_)),
                  full((D_, H_ * D_)), full((1, H_ * D_)),
                  full((D_, H_ * D_)), full((D_, H_ * D_)),
                  full((DE_, H_ * D_)), full((1, H_ * D_)),
                  full((1, H_ * D_))],
        out_specs=[row((BN, D_)), row((BN, D_)), row((BN, H_ * D_)),
                   full((D_, H_ * D_)), full((D_, H_ * DE_)),
                   full((H_ * DE_, D_)), full((D_, H_)), full((D_, H_)),
                   full((DE_, H_)), full((1, H_))],
        out_shape=[jax.ShapeDtypeStruct((N_, D_), jnp.float32),
                   jax.ShapeDtypeStruct((N_, D_), jnp.bfloat16),
                   jax.ShapeDtypeStruct((N_, H_ * D_), jnp.bfloat16),
                   jax.ShapeDtypeStruct((D_, H_ * D_), jnp.bfloat16),
                   jax.ShapeDtypeStruct((D_, H_ * DE_), jnp.bfloat16),
                   jax.ShapeDtypeStruct((H_ * DE_, D_), jnp.float32),
                   jax.ShapeDtypeStruct((D_, H_), jnp.float32),
                   jax.ShapeDtypeStruct((D_, H_), jnp.float32),
                   jax.ShapeDtypeStruct((DE_, H_), jnp.float32),
                   jax.ShapeDtypeStruct((1, H_), jnp.float32)],
    )(h, lnw2, lnb2, Wv, bv2, Wq, Wk, We, bq2, bk2)


_BE = 1024


def _alpha_body(hnd_ref, hns_ref, ea_ref, acat, pcat, avec, bvec, gvec, cconst,
                out_ref):
    hnd = hnd_ref[...]
    hns = hns_ref[...]
    hndf = hnd.astype(jnp.float32)
    hnsf = hns.astype(jnp.float32)
    ea = ea_ref[...]
    t1 = jnp.dot(hnd, acat[...], preferred_element_type=jnp.float32)
    hns_rep = jnp.concatenate([hnsf] * H_, axis=1)
    r1 = lax.broadcasted_iota(jnp.int32, (H_ * D_, H_), 0)
    c1 = lax.broadcasted_iota(jnp.int32, (H_ * D_, H_), 1)
    s1 = ((r1 // D_) == c1).astype(jnp.bfloat16)
    al = jnp.dot((t1 * hns_rep).astype(jnp.bfloat16), s1,
                 preferred_element_type=jnp.float32)
    t2 = jnp.dot(hnd, pcat[...], preferred_element_type=jnp.float32)
    ea_rep = jnp.concatenate([ea] * H_, axis=1)
    r2 = lax.broadcasted_iota(jnp.int32, (H_ * DE_, H_), 0)
    c2 = lax.broadcasted_iota(jnp.int32, (H_ * DE_, H_), 1)
    s2 = ((r2 // DE_) == c2).astype(jnp.bfloat16)
    al = al + jnp.dot((t2 * ea_rep).astype(jnp.bfloat16), s2,
                      preferred_element_type=jnp.float32)
    al = (al
          + jnp.dot(hndf, avec[...], preferred_element_type=jnp.float32)
          + jnp.dot(hnsf, bvec[...], preferred_element_type=jnp.float32)
          + jnp.dot(ea, gvec[...], preferred_element_type=jnp.float32)
          + cconst[...])
    al = al * RSQD
    al = jnp.concatenate([al, jnp.full((_BE, H_), NEG, jnp.float32)], axis=1)
    i = pl.program_id(0)
    rowid = i * _BE + lax.broadcasted_iota(jnp.int32, (_BE, 1), 0)
    out_ref[...] = jnp.where(rowid < E_, al, NEG)


def _alpha_call(hnd, hns, eap, acat, pcat, avec, bvec, gvec, cconst):
    grid = (EPAD // _BE,)
    row = lambda shape: pl.BlockSpec(shape, lambda i: (i, 0))
    full = lambda shape: pl.BlockSpec(shape, lambda i: (0, 0))
    return pl.pallas_call(
        _alpha_body,
        grid=grid,
        in_specs=[row((_BE, D_)), row((_BE, D_)), row((_BE, DE_)),
                  full((D_, H_ * D_)), full((D_, H_ * DE_)),
                  full((D_, H_)), full((D_, H_)), full((DE_, H_)),
                  full((1, H_))],
        out_specs=row((_BE, 2 * H_)),
        out_shape=jax.ShapeDtypeStruct((EPAD, 2 * H_), jnp.float32),
    )(hnd, hns, eap, acat, pcat, avec, bvec, gvec, cconst)


def _norm_body(acc0, acc1, wecat, invd_ref, ec_ref):
    den = acc0[:, D_:D_ + H_] + acc1[:, D_:D_ + H_]
    inv = 1.0 / (den + 1e-16)
    t = acc0[:, 0:D_] + acc1[:, 0:D_]
    parts = [t[:, h * DE_:(h + 1) * DE_] * inv[:, h:h + 1] for h in range(H_)]
    ts = jnp.concatenate(parts, axis=1)
    ec_ref[...] = jnp.dot(ts, wecat[...], preferred_element_type=jnp.float32)
    invd_ref[...] = jnp.concatenate([inv, jnp.zeros_like(inv)], axis=1)


def _norm_call(acc0, acc1, wecat):
    BN = 512
    grid = (NPAD // BN,)
    row = lambda shape: pl.BlockSpec(shape, lambda i: (i, 0))
    full = lambda shape: pl.BlockSpec(shape, lambda i: (0, 0))
    return pl.pallas_call(
        _norm_body,
        grid=grid,
        in_specs=[row((BN, ACCW)), row((BN, ACCW)), full((H_ * DE_, D_))],
        out_specs=[row((BN, 2 * H_)), row((BN, D_))],
        out_shape=[jax.ShapeDtypeStruct((NPAD, 2 * H_), jnp.float32),
                   jax.ShapeDtypeStruct((NPAD, D_), jnp.float32)],
    )(acc0, acc1, wecat)


def _final_body(hn_ref, o0_ref, o1_ref, ec_ref, wskip, bskip,
                gaW1, gab1, gaW2, gab2, gaW3r, gab3,
                ln2w, ln2b, ffW1, ffb1, ffW2, ffb2,
                gfW1, gfb1, gfW2, gfb2, gfW3r, gfb3, out_ref):
    hn = hn_ref[...]
    p = lax.broadcasted_iota(jnp.int32, (D_, D_), 0)
    c = lax.broadcasted_iota(jnp.int32, (D_, D_), 1)
    tgt = ((p >> 5) << 5) + 2 * (p & 15) + ((p >> 4) & 1)
    perm = (c == tgt).astype(jnp.float32)
    op = jnp.dot(o0_ref[...] + o1_ref[...], perm,
                 preferred_element_type=jnp.float32)
    out = ((op + ec_ref[...]) * (1.0 / H_)
           + jnp.dot(hn, wskip[...], preferred_element_type=jnp.float32)
           + bskip[...])

    def gate(u, v, W1, b1, W2, b2, W3r, b3):
        z = jnp.concatenate([u, v, u - v], axis=1)
        a = jnp.dot(z, W1[...], preferred_element_type=jnp.float32) + b1[...]
        a = a * jax.nn.sigmoid(a)
        a = jnp.dot(a, W2[...], preferred_element_type=jnp.float32) + b2[...]
        a = a * jax.nn.sigmoid(a)
        g = jnp.sum(a * W3r[...], axis=1, keepdims=True) + b3[...]
        g = jax.nn.sigmoid(g)
        return g * u + (1 - g) * v

    h1 = gate(hn, out, gaW1, gab1, gaW2, gab2, gaW3r, gab3)
    mu = jnp.mean(h1, axis=1, keepdims=True)
    var = jnp.mean((h1 - mu) ** 2, axis=1, keepdims=True)
    h2 = (h1 - mu) / jnp.sqrt(var + 1e-5) * ln2w[...] + ln2b[...]
    ff = jnp.dot(h2, ffW1[...], preferred_element_type=jnp.float32) + ffb1[...]
    ff = ff * jax.nn.sigmoid(ff)
    ff = jnp.dot(ff, ffW2[...], preferred_element_type=jnp.float32) + ffb2[...]
    out_ref[...] = gate(h2, ff, gfW1, gfb1, gfW2, gfb2, gfW3r, gfb3)


def _final_call(hn, o0, o1, ec, Wskip, bskip2, ga, ln2w2, ln2b2, ff, gf):
    BN = 400
    grid = (N_ // BN,)
    row = lambda shape: pl.BlockSpec(shape, lambda i: (i, 0))
    full = lambda shape: pl.BlockSpec(shape, lambda i: (0, 0))
    D3, D32, D34 = 3 * D_, 3 * D_ // 2, 3 * D_ // 4
    in_specs = [row((BN, D_)), row((BN, D_)), row((BN, D_)), row((BN, D_)),
                full((D_, D_)), full((1, D_)),
                full((D3, D32)), full((1, D32)), full((D32, D34)), full((1, D34)),
                full((1, D34)), full((1, 1)),
                full((1, D_)), full((1, D_)),
                full((D_, D_)), full((1, D_)), full((D_, D_)), full((1, D_)),
                full((D3, D32)), full((1, D32)), full((D32, D34)), full((1, D34)),
                full((1, D34)), full((1, 1))]
    return pl.pallas_call(
        _final_body,
        grid=grid,
        in_specs=in_specs,
        out_specs=row((BN, D_)),
        out_shape=jax.ShapeDtypeStruct((N_, D_), jnp.float32),
    )(hn, o0, o1, ec, Wskip, bskip2, *ga, ln2w2, ln2b2, *ff, *gf)


def _sc_gather(hnb, srcp, dstp):
    CB = 128
    nch = EPW // CB

    @functools.partial(
        pl.kernel,
        out_type=(jax.ShapeDtypeStruct((EPAD, D_), jnp.bfloat16),
                  jax.ShapeDtypeStruct((EPAD, D_), jnp.bfloat16)),
        mesh=_MESH(),
        compiler_params=pltpu.CompilerParams(use_tc_tiling_on_sc=False, needs_layout_passes=False),
        scratch_types=[pltpu.VMEM((EPW // 128, 128), jnp.int32),
                       pltpu.VMEM((EPW // 128, 128), jnp.int32),
                       pltpu.VMEM((2, CB, D_), jnp.bfloat16),
                       pltpu.VMEM((2, CB, D_), jnp.bfloat16),
                       pltpu.SemaphoreType.DMA, pltpu.SemaphoreType.DMA,
                       pltpu.SemaphoreType.DMA, pltpu.SemaphoreType.DMA,
                       pltpu.SemaphoreType.DMA, pltpu.SemaphoreType.DMA,
                       pltpu.SemaphoreType.DMA, pltpu.SemaphoreType.DMA],
    )
    def k(hn_hbm, src_hbm, dst_hbm, hns_hbm, hnd_hbm,
          sidx, didx, srows, drows,
          s_s0, s_s1, s_d0, s_d1, w_s0, w_s1, w_d0, w_d1):
        wid = lax.axis_index("s") * NC + lax.axis_index("c")
        ssems = (s_s0, s_s1)
        dsems = (s_d0, s_d1)
        wssems = (w_s0, w_s1)
        wdsems = (w_d0, w_d1)

        pltpu.sync_copy(src_hbm.at[pl.ds(wid * nch, nch)], sidx)
        pltpu.sync_copy(dst_hbm.at[pl.ds(wid * nch, nch)], didx)

        def start(j, b):
            pltpu.async_copy(hn_hbm.at[sidx.at[j]], srows.at[b], ssems[b])
            pltpu.async_copy(hn_hbm.at[didx.at[j]], drows.at[b], dsems[b])

        def waitwb(b):
            base = wid * EPW
            pltpu.make_async_copy(srows.at[b], hns_hbm.at[pl.ds(base, CB)],
                                  wssems[b]).wait()
            pltpu.make_async_copy(drows.at[b], hnd_hbm.at[pl.ds(base, CB)],
                                  wdsems[b]).wait()

        def drain(j, b):
            base = wid * EPW + j * CB
            pltpu.make_async_copy(hn_hbm.at[sidx.at[j]], srows.at[b],
                                  ssems[b]).wait()
            pltpu.make_async_copy(hn_hbm.at[didx.at[j]], drows.at[b],
                                  dsems[b]).wait()
            pltpu.async_copy(srows.at[b], hns_hbm.at[pl.ds(base, CB)],
                             wssems[b])
            pltpu.async_copy(drows.at[b], hnd_hbm.at[pl.ds(base, CB)],
                             wdsems[b])

        start(0, 0)
        start(1, 1)

        def body(p, carry):
            drain(2 * p, 0)

            @pl.when(p + 1 < nch // 2)
            def _():
                waitwb(0)
                start(2 * p + 2, 0)
            drain(2 * p + 1, 1)

            @pl.when(p + 1 < nch // 2)
            def _():
                waitwb(1)
                start(2 * p + 3, 1)
            return carry

        lax.fori_loop(0, nch // 2, body, 0)
        waitwb(0)
        waitwb(1)

    return k(hnb, srcp.reshape(EPAD // CB, CB), dstp.reshape(EPAD // CB, CB))


def _sc_p1(alpha, eap, dstp):
    CB = 64
    nch = EPW // CB
    rpt = NPAD // NS

    @functools.partial(
        pl.kernel,
        out_type=(jax.ShapeDtypeStruct((NPAD, ACCW), jnp.float32),
                  jax.ShapeDtypeStruct((NPAD, ACCW), jnp.float32)),
        mesh=_MESH(),
        compiler_params=pltpu.CompilerParams(use_tc_tiling_on_sc=False, needs_layout_passes=False),
        scratch_types=[pltpu.VMEM((2, CB, 2 * H_), jnp.float32),
                       pltpu.VMEM((2, CB, DE_), jnp.float32),
                       pltpu.VMEM((nch, CB), jnp.int32),
                       pltpu.VMEM((2, CB, ACCW), jnp.float32),
                       pltpu.VMEM_SHARED((NPAD, ACCW), jnp.float32),
                       pltpu.SemaphoreType.DMA, pltpu.SemaphoreType.DMA,
                       pltpu.SemaphoreType.DMA, pltpu.SemaphoreType.DMA,
                       pltpu.SemaphoreType.DMA, pltpu.SemaphoreType.DMA],
    )
    def k(alpha_hbm, ea_hbm, dst_hbm, out0_hbm, out1_hbm,
          abuf, eabuf, didx, payload, acc, sa0, sa1, se0, se1, sc0, sc1):
        cid = lax.axis_index("c")
        sid = lax.axis_index("s")
        wid = sid * NC + cid
        asems = (sa0, sa1)
        esems = (se0, se1)
        csems = (sc0, sc1)

        pltpu.sync_copy(dst_hbm.at[pl.ds(wid * nch, nch)], didx)

        def zrow(i, c):
            for bb in range(2):
                for j in range(ACCW // 16):
                    payload[bb, i, pl.ds(j * 16, 16)] = jnp.zeros(
                        (16,), jnp.float32)
            return c
        lax.fori_loop(0, CB, zrow, 0)

        def zacc(i, c):
            pltpu.sync_copy(payload.at[0],
                            acc.at[pl.ds(sid * rpt + i * CB, CB)])
            return c
        lax.fori_loop(0, rpt // CB, zacc, 0)
        plsc.subcore_barrier()

        def start(j, b):
            base = wid * EPW + j * CB
            pltpu.async_copy(alpha_hbm.at[pl.ds(base, CB)], abuf.at[b],
                             asems[b])
            pltpu.async_copy(ea_hbm.at[pl.ds(base, CB)], eabuf.at[b],
                             esems[b])

        def waitsc(j, b):
            pltpu.make_async_copy(payload.at[b], acc.at[didx.at[j]],
                                  csems[b]).wait()

        def process(j, b):
            base = wid * EPW + j * CB
            pltpu.make_async_copy(alpha_hbm.at[pl.ds(base, CB)], abuf.at[b],
                                  asems[b]).wait()
            pltpu.make_async_copy(ea_hbm.at[pl.ds(base, CB)], eabuf.at[b],
                                  esems[b]).wait()

            def edge(e, c2):
                ex16 = jnp.exp(abuf[b, e, pl.ds(0, 16)])
                payload[b, e, pl.ds(D_, 16)] = ex16
                earow = eabuf[b, e, pl.ds(0, DE_)]
                for h in range(H_):
                    payload[b, e, pl.ds(h * DE_, DE_)] = (
                        jnp.full((16,), ex16[h]) * earow)
                return c2

            lax.fori_loop(0, CB, edge, 0)
            pltpu.async_copy(payload.at[b], acc.at[didx.at[j]], csems[b],
                             add=True)

        start(0, 0)
        start(1, 1)

        def body(p, carry):
            process(2 * p, 0)

            @pl.when(p + 1 < nch // 2)
            def _():
                start(2 * p + 2, 0)
            process(2 * p + 1, 1)

            @pl.when(p + 1 < nch // 2)
            def _():
                start(2 * p + 3, 1)

            @pl.when(p + 1 < nch // 2)
            def _():
                waitsc(2 * p, 0)
                waitsc(2 * p + 1, 1)
            return carry

        lax.fori_loop(0, nch // 2, body, 0)
        waitsc(nch - 2, 0)
        waitsc(nch - 1, 1)
        plsc.subcore_barrier()

        def wout(i, c):
            r0 = sid * rpt + i * CB

            @pl.when(cid == 0)
            def _():
                pltpu.sync_copy(acc.at[pl.ds(r0, CB)],
                                out0_hbm.at[pl.ds(r0, CB)])

            @pl.when(cid == 1)
            def _():
                pltpu.sync_copy(acc.at[pl.ds(r0, CB)],
                                out1_hbm.at[pl.ds(r0, CB)])
            return c
        lax.fori_loop(0, rpt // CB, wout, 0)

    return k(alpha, eap, dstp.reshape(EPAD // CB, CB))


def _sc_p2(alpha, srcp, dstp, Vb, invd):
    CB = 32
    nch = EPW // CB
    rpt = NPAD // NS

    @functools.partial(
        pl.kernel,
        out_type=(jax.ShapeDtypeStruct((NPAD, D_), jnp.float32),
                  jax.ShapeDtypeStruct((NPAD, D_), jnp.float32)),
        mesh=_MESH(),
        compiler_params=pltpu.CompilerParams(use_tc_tiling_on_sc=False, needs_layout_passes=False),
        scratch_types=[pltpu.VMEM((2, CB, 2 * H_), jnp.float32),
                       pltpu.VMEM((2, CB, 2 * H_), jnp.float32),
                       pltpu.VMEM((nch, CB), jnp.int32),
                       pltpu.VMEM((nch, CB), jnp.int32),
                       pltpu.VMEM((2, CB, H_ * D_), jnp.bfloat16),
                       pltpu.VMEM((CB, D_), jnp.float32),
                       pltpu.VMEM_SHARED((NPAD, D_), jnp.float32),
                       pltpu.SemaphoreType.DMA, pltpu.SemaphoreType.DMA,
                       pltpu.SemaphoreType.DMA, pltpu.SemaphoreType.DMA],
    )
    def k(alpha_hbm, src_hbm, dst_hbm, v_hbm, invd_hbm, out0_hbm, out1_hbm,
          abuf, ivbuf, sidx, didx, vrows, wpay, acc, sv0, sv1, si0, si1):
        cid = lax.axis_index("c")
        sid = lax.axis_index("s")
        wid = sid * NC + cid
        vsems = (sv0, sv1)
        isems = (si0, si1)

        pltpu.sync_copy(src_hbm.at[pl.ds(wid * nch, nch)], sidx)
        pltpu.sync_copy(dst_hbm.at[pl.ds(wid * nch, nch)], didx)

        def zrow(i, c):
            for j in range(D_ // 16):
                wpay[i, pl.ds(j * 16, 16)] = jnp.zeros((16,), jnp.float32)
            return c
        lax.fori_loop(0, CB, zrow, 0)

        def zacc(i, c):
            pltpu.sync_copy(wpay, acc.at[pl.ds(sid * rpt + i * CB, CB)])
            return c
        lax.fori_loop(0, rpt // CB, zacc, 0)
        plsc.subcore_barrier()

        def start(j, b):
            base = wid * EPW + j * CB
            pltpu.sync_copy(alpha_hbm.at[pl.ds(base, CB)], abuf.at[b])
            pltpu.async_copy(v_hbm.at[sidx.at[j]], vrows.at[b], vsems[b])
            pltpu.async_copy(invd_hbm.at[didx.at[j]], ivbuf.at[b], isems[b])

        def process(j, b):
            pltpu.make_async_copy(
                v_hbm.at[sidx.at[j]], vrows.at[b], vsems[b]).wait()
            pltpu.make_async_copy(
                invd_hbm.at[didx.at[j]], ivbuf.at[b], isems[b]).wait()

            def edge(e, c2):
                attn16 = (jnp.exp(abuf[b, e, pl.ds(0, 16)])
                          * ivbuf[b, e, pl.ds(0, 16)])
                accs = [jnp.zeros((16,), jnp.float32)
                        for _ in range(D_ // 16)]
                for h in range(H_):
                    avf = jnp.full((16,), attn16[h])
                    avv = plsc.pack(avf, avf,
                                    format=plsc.PackFormat.INTERLEAVED)
                    for g in range(D_ // 32):
                        x32 = vrows[b, e, pl.ds(h * D_ + g * 32, 32)]
                        lo, hi = plsc.unpack(
                            x32 * avv, format=plsc.PackFormat.INTERLEAVED)
                        accs[2 * g] = accs[2 * g] + lo
                        accs[2 * g + 1] = accs[2 * g + 1] + hi
                for dv in range(D_ // 16):
                    wpay[e, pl.ds(dv * 16, 16)] = accs[dv]
                return c2

            lax.fori_loop(0, CB, edge, 0)
            pltpu.sync_copy(wpay, acc.at[didx.at[j]], add=True)

        start(0, 0)

        def body(p, carry):
            start(2 * p + 1, 1)
            process(2 * p, 0)

            @pl.when(p + 1 < nch // 2)
            def _():
                start(2 * p + 2, 0)
            process(2 * p + 1, 1)
            return carry

        lax.fori_loop(0, nch // 2, body, 0)
        plsc.subcore_barrier()

        def wout(i, c):
            r0 = sid * rpt + i * CB

            @pl.when(cid == 0)
            def _():
                pltpu.sync_copy(acc.at[pl.ds(r0, CB)],
                                out0_hbm.at[pl.ds(r0, CB)])

            @pl.when(cid == 1)
            def _():
                pltpu.sync_copy(acc.at[pl.ds(r0, CB)],
                                out1_hbm.at[pl.ds(r0, CB)])
            return c
        lax.fori_loop(0, rpt // CB, wout, 0)

    return k(alpha, srcp.reshape(EPAD // CB, CB), dstp.reshape(EPAD // CB, CB),
             Vb, invd)


def kernel(h, edge_index, edge_attr, ln1_w, ln1_b, Wq, bq, Wk, bk, Wv, bv, We,
           Wskip, bskip, ga_W1, ga_b1, ga_W2, ga_b2, ga_W3, ga_b3, ln2_w, ln2_b,
           ff_W1, ff_b1, ff_W2, ff_b2, gf_W1, gf_b1, gf_W2, gf_b2, gf_W3, gf_b3):
    pad_e = EPAD - E_
    srcp = jnp.concatenate([edge_index[0], jnp.zeros((pad_e,), jnp.int32)])
    dstp = jnp.concatenate([edge_index[1], jnp.zeros((pad_e,), jnp.int32)])
    eap = jnp.concatenate(
        [edge_attr, jnp.zeros((pad_e, DE_), jnp.float32)], axis=0)

    (hn, hnb, Vb, acat, pcat, wecat, avec, bvec, gvec, cconst) = _hnv_call(
        h, ln1_w.reshape(1, -1), ln1_b.reshape(1, -1), Wv, bv.reshape(1, -1),
        Wq, Wk, We, bq.reshape(1, -1), bk.reshape(1, -1))
    hns, hnd = _sc_gather(hnb, srcp, dstp)
    alpha = _alpha_call(hnd, hns, eap, acat, pcat, avec, bvec, gvec, cconst)
    acc0, acc1 = _sc_p1(alpha, eap, dstp)
    invd, ec = _norm_call(acc0, acc1, wecat)
    o0, o1 = _sc_p2(alpha, srcp, dstp, Vb, invd)
    ga = (ga_W1, ga_b1.reshape(1, -1), ga_W2, ga_b2.reshape(1, -1),
          ga_W3.reshape(1, -1), ga_b3.reshape(1, -1))
    ff = (ff_W1, ff_b1.reshape(1, -1), ff_W2, ff_b2.reshape(1, -1))
    gf = (gf_W1, gf_b1.reshape(1, -1), gf_W2, gf_b2.reshape(1, -1),
          gf_W3.reshape(1, -1), gf_b3.reshape(1, -1))
    return _final_call(hn, o0, o1, ec,
                       Wskip, bskip.reshape(1, -1), ga,
                       ln2_w.reshape(1, -1), ln2_b.reshape(1, -1), ff, gf)
